# baseline (speedup 1.0000x reference)
import functools

import jax
import jax.numpy as jnp
from jax import lax
from jax.experimental import pallas as pl
from jax.experimental.pallas import tpu as pltpu
from jax.experimental.pallas import tpu_sc as plsc

N = 10000
NP = 10240
E = 320000
EP = 327680
D_IN = 128
D_H = 128
D_OUT = 64

NC = 2
NS = 16
NW = NC * NS
EPW = EP // NW
RPT = NP // NS
NR = 4


def _sc_mesh():
    return plsc.VectorSubcoreMesh(core_axis_name="c", subcore_axis_name="s")


_SC_PARAMS = pltpu.CompilerParams(use_tc_tiling_on_sc=False)


DEG_K = 128
DEG_NCH = EPW // DEG_K


def _make_deg_kernel():
    @functools.partial(
        pl.kernel,
        out_type=(
            jax.ShapeDtypeStruct((NC, NP), jnp.float32),
            jax.ShapeDtypeStruct((NC, NP), jnp.float32),
        ),
        mesh=_sc_mesh(),
        scratch_types=(
            [pltpu.VMEM((DEG_K,), jnp.int32) for _ in range(2 * NR)]
            + [pltpu.VMEM((DEG_K,), jnp.float32),
               pltpu.VMEM_SHARED((NP,), jnp.float32),
               pltpu.VMEM_SHARED((NP,), jnp.float32)]
            + [pltpu.SemaphoreType.DMA for _ in range(NR + 1)]
        ),
        compiler_params=_SC_PARAMS,
    )
    def deg_kernel(src_hbm, dst_hbm, zeros_hbm, dout_hbm, din_hbm, *refs):
        sidx = refs[0:NR]
        didx = refs[NR:2 * NR]
        ones_v, dsrc_sh, ddst_sh = refs[2 * NR:2 * NR + 3]
        isem = refs[2 * NR + 3:2 * NR + 3 + NR]
        ssem = refs[2 * NR + 3 + NR]
        c = lax.axis_index("c")
        s = lax.axis_index("s")
        wid = c * NS + s
        base_n = s * RPT
        pltpu.sync_copy(zeros_hbm.at[pl.ds(base_n, RPT)],
                        dsrc_sh.at[pl.ds(base_n, RPT)])
        pltpu.sync_copy(zeros_hbm.at[pl.ds(base_n, RPT)],
                        ddst_sh.at[pl.ds(base_n, RPT)])
        for i in range(DEG_K // 16):
            ones_v[pl.ds(i * 16, 16)] = jnp.ones((16,), jnp.float32)
        plsc.subcore_barrier()

        def start_idx(j, m):
            base_e = wid * EPW + jnp.minimum(j, DEG_NCH - 1) * DEG_K
            pltpu.async_copy(src_hbm.at[pl.ds(base_e, DEG_K)], sidx[m],
                             isem[m])
            pltpu.async_copy(dst_hbm.at[pl.ds(base_e, DEG_K)], didx[m],
                             isem[m])

        def wait_idx(m):
            pltpu.make_async_copy(src_hbm.at[pl.ds(0, DEG_K)], sidx[m],
                                  isem[m]).wait()
            pltpu.make_async_copy(dst_hbm.at[pl.ds(0, DEG_K)], didx[m],
                                  isem[m]).wait()

        start_idx(0, 0)
        start_idx(1, 1)

        def body(g, carry):
            for p in range(NR):
                j = g * NR + p
                wait_idx(p)
                start_idx(j + 2, (p + 2) % NR)
                pltpu.async_copy(ones_v, dsrc_sh.at[sidx[p]], ssem, add=True)
                pltpu.sync_copy(ones_v, ddst_sh.at[didx[p]], add=True)
                pltpu.make_async_copy(zeros_hbm.at[pl.ds(0, DEG_K)], ones_v,
                                      ssem).wait()
            return carry

        lax.fori_loop(0, DEG_NCH // NR, body, 0)
        wait_idx(DEG_NCH % NR)
        wait_idx((DEG_NCH + 1) % NR)
        plsc.subcore_barrier()
        pltpu.sync_copy(dsrc_sh.at[pl.ds(base_n, RPT)],
                        dout_hbm.at[c, pl.ds(base_n, RPT)])
        pltpu.sync_copy(ddst_sh.at[pl.ds(base_n, RPT)],
                        din_hbm.at[c, pl.ds(base_n, RPT)])

    return deg_kernel


def _make_agg_kernel(d, k):
    nch = EPW // k
    assert nch % NR == 0

    @functools.partial(
        pl.kernel,
        out_type=jax.ShapeDtypeStruct((NC, NP, d), jnp.float32),
        mesh=_sc_mesh(),
        scratch_types=(
            [pltpu.VMEM((k,), jnp.int32) for _ in range(2 * NR)]
            + [pltpu.VMEM((k, d), jnp.float32) for _ in range(NR)]
            + [pltpu.VMEM_SHARED((NP, d), jnp.float32)]
            + [pltpu.SemaphoreType.DMA for _ in range(3 * NR)]
        ),
        compiler_params=_SC_PARAMS,
    )
    def agg_kernel(src_hbm, dst_hbm, h_hbm, zeros_hbm, out_hbm, *refs):
        sidx = refs[0:NR]
        didx = refs[NR:2 * NR]
        rows = refs[2 * NR:3 * NR]
        agg_sh = refs[3 * NR]
        isem = refs[3 * NR + 1:3 * NR + 1 + NR]
        gsem = refs[3 * NR + 1 + NR:3 * NR + 1 + 2 * NR]
        ssem = refs[3 * NR + 1 + 2 * NR:3 * NR + 1 + 3 * NR]
        c = lax.axis_index("c")
        s = lax.axis_index("s")
        wid = c * NS + s
        base_n = s * RPT
        pltpu.sync_copy(zeros_hbm.at[pl.ds(base_n, RPT)],
                        agg_sh.at[pl.ds(base_n, RPT)])

        def start_idx(j, m):
            base_e = wid * EPW + jnp.minimum(j, nch - 1) * k
            pltpu.async_copy(src_hbm.at[pl.ds(base_e, k)], sidx[m], isem[m])
            pltpu.async_copy(dst_hbm.at[pl.ds(base_e, k)], didx[m], isem[m])

        def wait_idx(m):
            pltpu.make_async_copy(src_hbm.at[pl.ds(0, k)], sidx[m],
                                  isem[m]).wait()
            pltpu.make_async_copy(dst_hbm.at[pl.ds(0, k)], didx[m],
                                  isem[m]).wait()

        def start_gather(m):
            pltpu.async_copy(h_hbm.at[sidx[m]], rows[m], gsem[m])

        def wait_gather(m):
            pltpu.make_async_copy(h_hbm.at[pl.ds(0, k)], rows[m],
                                  gsem[m]).wait()

        def start_scatter(m):
            pltpu.async_copy(rows[m], agg_sh.at[didx[m]], ssem[m], add=True)

        def wait_scatter(m):
            pltpu.make_async_copy(h_hbm.at[pl.ds(0, k)], rows[m],
                                  ssem[m]).wait()

        def phase(j, p):
            p1 = (p + 1) % NR
            p2 = (p + 2) % NR
            wait_gather(p)
            if not (isinstance(j, int) and j < 2):
                wait_scatter(p2)
            wait_idx(p1)
            start_gather(p1)
            start_idx(j + 2, p2)
            start_scatter(p)

        plsc.subcore_barrier()
        start_idx(0, 0)
        start_idx(1, 1)
        wait_idx(0)
        start_gather(0)
        for j in range(NR):
            phase(j, j)

        def body(g, carry):
            for p in range(NR):
                phase(NR + g * NR + p, p)
            return carry

        lax.fori_loop(0, (nch - NR) // NR, body, 0)
        wait_gather(nch % NR)
        wait_idx((nch + 1) % NR)
        wait_scatter((nch - 2) % NR)
        wait_scatter((nch - 1) % NR)
        plsc.subcore_barrier()
        pltpu.sync_copy(agg_sh.at[pl.ds(base_n, RPT)],
                        out_hbm.at[c, pl.ds(base_n, RPT)])

    return agg_kernel


BLK = 2048


def _layer1_body(dout_ref, din_ref, x_ref, w_ref, so_ref, si_ref, h_ref):
    deg_out = jnp.maximum(dout_ref[0, :] + dout_ref[1, :], 1.0)
    deg_in = jnp.maximum(din_ref[0, :] + din_ref[1, :], 1.0)
    so = lax.rsqrt(deg_out)
    si = lax.rsqrt(deg_in)
    so_ref[...] = so
    si_ref[...] = si
    h_ref[...] = jnp.dot(x_ref[...] * so[:, None], w_ref[...],
                         preferred_element_type=jnp.float32)


def _tc_layer1(dout_p, din_p, x_pad, W1):
    return pl.pallas_call(
        _layer1_body,
        grid=(NP // BLK,),
        in_specs=[
            pl.BlockSpec((NC, BLK), lambda i: (0, i)),
            pl.BlockSpec((NC, BLK), lambda i: (0, i)),
            pl.BlockSpec((BLK, D_IN), lambda i: (i, 0)),
            pl.BlockSpec((D_IN, D_H), lambda i: (0, 0)),
        ],
        out_specs=[
            pl.BlockSpec((BLK,), lambda i: (i,)),
            pl.BlockSpec((BLK,), lambda i: (i,)),
            pl.BlockSpec((BLK, D_H), lambda i: (i, 0)),
        ],
        out_shape=[
            jax.ShapeDtypeStruct((NP,), jnp.float32),
            jax.ShapeDtypeStruct((NP,), jnp.float32),
            jax.ShapeDtypeStruct((NP, D_H), jnp.float32),
        ],
    )(dout_p, din_p, x_pad, W1)


def _layer2_body(agg_ref, si_ref, so_ref, b_ref, w_ref, h_ref):
    agg = agg_ref[0, :, :] + agg_ref[1, :, :]
    h = agg * si_ref[...][:, None] + b_ref[...][None, :]
    h = jnp.maximum(h, 0.0)
    h_ref[...] = jnp.dot(h * so_ref[...][:, None], w_ref[...],
                         preferred_element_type=jnp.float32)


def _tc_layer2(aggp1, si, so, b1, W2):
    return pl.pallas_call(
        _layer2_body,
        grid=(NP // BLK,),
        in_specs=[
            pl.BlockSpec((NC, BLK, D_H), lambda i: (0, i, 0)),
            pl.BlockSpec((BLK,), lambda i: (i,)),
            pl.BlockSpec((BLK,), lambda i: (i,)),
            pl.BlockSpec((D_H,), lambda i: (0,)),
            pl.BlockSpec((D_H, D_OUT), lambda i: (0, 0)),
        ],
        out_specs=pl.BlockSpec((BLK, D_OUT), lambda i: (i, 0)),
        out_shape=jax.ShapeDtypeStruct((NP, D_OUT), jnp.float32),
    )(aggp1, si, so, b1, W2)


def _final_body(agg_ref, si_ref, b_ref, out_ref):
    agg = agg_ref[0, :, :] + agg_ref[1, :, :]
    out_ref[...] = agg * si_ref[...][:, None] + b_ref[...][None, :]


def _tc_final(aggp2, si, b2):
    return pl.pallas_call(
        _final_body,
        grid=(NP // BLK,),
        in_specs=[
            pl.BlockSpec((NC, BLK, D_OUT), lambda i: (0, i, 0)),
            pl.BlockSpec((BLK,), lambda i: (i,)),
            pl.BlockSpec((D_OUT,), lambda i: (0,)),
        ],
        out_specs=pl.BlockSpec((BLK, D_OUT), lambda i: (i, 0)),
        out_shape=jax.ShapeDtypeStruct((NP, D_OUT), jnp.float32),
    )(aggp2, si, b2)


def kernel(features, edge_index, W1, b1, W2, b2):
    epad = jnp.full((2, EP - E), NP - 1, jnp.int32)
    ei = jnp.concatenate([edge_index, epad], axis=1)
    src = ei[0]
    dst = ei[1]
    x_pad = jnp.pad(features, ((0, NP - N), (0, 0)))
    zeros_n = jnp.zeros((NP,), jnp.float32)
    zeros_h = jnp.zeros((NP, D_H), jnp.float32)
    zeros_o = jnp.zeros((NP, D_OUT), jnp.float32)

    dout_p, din_p = _make_deg_kernel()(src, dst, zeros_n)
    so, si, h1 = _tc_layer1(dout_p, din_p, x_pad, W1)
    aggp1 = _make_agg_kernel(D_H, 80)(src, dst, h1, zeros_h)
    h2 = _tc_layer2(aggp1, si, so, b1, W2)
    aggp2 = _make_agg_kernel(D_OUT, 128)(src, dst, h2, zeros_o)
    return _tc_final(aggp2, si, b2)[:N]


# speedup vs baseline: 2.2211x; 2.2211x over previous
import functools

import jax
import jax.numpy as jnp
from jax import lax
from jax.experimental import pallas as pl
from jax.experimental.pallas import tpu as pltpu
from jax.experimental.pallas import tpu_sc as plsc

N = 10000
NP = 10240
E = 320000
EP = 327680
D_IN = 128
D_H = 128
D_OUT = 64

NC = 2
NS = 16
NW = NC * NS
EPW = EP // NW
RPT = NP // NS
NR = 4


def _sc_mesh():
    return plsc.VectorSubcoreMesh(core_axis_name="c", subcore_axis_name="s")


_SC_PARAMS = pltpu.CompilerParams(use_tc_tiling_on_sc=False)


DEG_K = 128
DEG_NCH = EPW // DEG_K


def _make_deg_kernel():
    @functools.partial(
        pl.kernel,
        out_type=(
            jax.ShapeDtypeStruct((NC, NP), jnp.float32),
            jax.ShapeDtypeStruct((NC, NP), jnp.float32),
        ),
        mesh=_sc_mesh(),
        scratch_types=(
            [pltpu.VMEM((DEG_K,), jnp.int32) for _ in range(2 * NR)]
            + [pltpu.VMEM((DEG_K,), jnp.float32),
               pltpu.VMEM_SHARED((NP,), jnp.float32),
               pltpu.VMEM_SHARED((NP,), jnp.float32)]
            + [pltpu.SemaphoreType.DMA for _ in range(NR + 1)]
        ),
        compiler_params=_SC_PARAMS,
    )
    def deg_kernel(src_hbm, dst_hbm, zeros_hbm, dout_hbm, din_hbm, *refs):
        sidx = refs[0:NR]
        didx = refs[NR:2 * NR]
        ones_v, dsrc_sh, ddst_sh = refs[2 * NR:2 * NR + 3]
        isem = refs[2 * NR + 3:2 * NR + 3 + NR]
        ssem = refs[2 * NR + 3 + NR]
        c = lax.axis_index("c")
        s = lax.axis_index("s")
        wid = c * NS + s
        base_n = s * RPT
        pltpu.sync_copy(zeros_hbm.at[pl.ds(base_n, RPT)],
                        dsrc_sh.at[pl.ds(base_n, RPT)])
        pltpu.sync_copy(zeros_hbm.at[pl.ds(base_n, RPT)],
                        ddst_sh.at[pl.ds(base_n, RPT)])
        for i in range(DEG_K // 16):
            ones_v[pl.ds(i * 16, 16)] = jnp.ones((16,), jnp.float32)
        plsc.subcore_barrier()

        def start_idx(j, m):
            base_e = wid * EPW + jnp.minimum(j, DEG_NCH - 1) * DEG_K
            pltpu.async_copy(src_hbm.at[pl.ds(base_e, DEG_K)], sidx[m],
                             isem[m])
            pltpu.async_copy(dst_hbm.at[pl.ds(base_e, DEG_K)], didx[m],
                             isem[m])

        def wait_idx(m):
            pltpu.make_async_copy(src_hbm.at[pl.ds(0, DEG_K)], sidx[m],
                                  isem[m]).wait()
            pltpu.make_async_copy(dst_hbm.at[pl.ds(0, DEG_K)], didx[m],
                                  isem[m]).wait()

        start_idx(0, 0)
        start_idx(1, 1)

        def body(g, carry):
            for p in range(NR):
                j = g * NR + p
                wait_idx(p)
                start_idx(j + 2, (p + 2) % NR)
                pltpu.async_copy(ones_v, dsrc_sh.at[sidx[p]], ssem, add=True)
                pltpu.sync_copy(ones_v, ddst_sh.at[didx[p]], add=True)
                pltpu.make_async_copy(zeros_hbm.at[pl.ds(0, DEG_K)], ones_v,
                                      ssem).wait()
            return carry

        lax.fori_loop(0, DEG_NCH // NR, body, 0)
        wait_idx(DEG_NCH % NR)
        wait_idx((DEG_NCH + 1) % NR)
        plsc.subcore_barrier()
        pltpu.sync_copy(dsrc_sh.at[pl.ds(base_n, RPT)],
                        dout_hbm.at[c, pl.ds(base_n, RPT)])
        pltpu.sync_copy(ddst_sh.at[pl.ds(base_n, RPT)],
                        din_hbm.at[c, pl.ds(base_n, RPT)])

    return deg_kernel


def _make_agg_kernel(d, k):
    nch = EPW // k
    assert nch % NR == 0

    @functools.partial(
        pl.kernel,
        out_type=jax.ShapeDtypeStruct((NC, NP, d), jnp.float32),
        mesh=_sc_mesh(),
        scratch_types=(
            [pltpu.VMEM((k,), jnp.int32) for _ in range(2 * NR)]
            + [pltpu.VMEM((k, d), jnp.float32) for _ in range(NR)]
            + [pltpu.VMEM_SHARED((NP, d), jnp.float32)]
            + [pltpu.SemaphoreType.DMA for _ in range(3 * NR)]
        ),
        compiler_params=_SC_PARAMS,
    )
    def agg_kernel(src_hbm, dst_hbm, h_hbm, zeros_hbm, out_hbm, *refs):
        sidx = refs[0:NR]
        didx = refs[NR:2 * NR]
        rows = refs[2 * NR:3 * NR]
        agg_sh = refs[3 * NR]
        isem = refs[3 * NR + 1:3 * NR + 1 + NR]
        gsem = refs[3 * NR + 1 + NR:3 * NR + 1 + 2 * NR]
        ssem = refs[3 * NR + 1 + 2 * NR:3 * NR + 1 + 3 * NR]
        c = lax.axis_index("c")
        s = lax.axis_index("s")
        wid = c * NS + s
        base_n = s * RPT
        pltpu.sync_copy(zeros_hbm.at[pl.ds(base_n, RPT)],
                        agg_sh.at[pl.ds(base_n, RPT)])

        def start_idx(j, m):
            base_e = wid * EPW + jnp.minimum(j, nch - 1) * k
            pltpu.async_copy(src_hbm.at[pl.ds(base_e, k)], sidx[m], isem[m])
            pltpu.async_copy(dst_hbm.at[pl.ds(base_e, k)], didx[m], isem[m])

        def wait_idx(m):
            pltpu.make_async_copy(src_hbm.at[pl.ds(0, k)], sidx[m],
                                  isem[m]).wait()
            pltpu.make_async_copy(dst_hbm.at[pl.ds(0, k)], didx[m],
                                  isem[m]).wait()

        def start_gather(m):
            pltpu.async_copy(h_hbm.at[sidx[m]], rows[m], gsem[m])

        def wait_gather(m):
            pltpu.make_async_copy(h_hbm.at[pl.ds(0, k)], rows[m],
                                  gsem[m]).wait()

        def start_scatter(m):
            pltpu.async_copy(rows[m], agg_sh.at[didx[m]], ssem[m], add=True)

        def wait_scatter(m):
            pltpu.make_async_copy(h_hbm.at[pl.ds(0, k)], rows[m],
                                  ssem[m]).wait()

        def phase(j, p):
            p1 = (p + 1) % NR
            p2 = (p + 2) % NR
            wait_gather(p)
            if not (isinstance(j, int) and j < 2):
                wait_scatter(p2)
            wait_idx(p1)
            start_gather(p1)
            start_idx(j + 2, p2)
            start_scatter(p)

        plsc.subcore_barrier()
        start_idx(0, 0)
        start_idx(1, 1)
        wait_idx(0)
        start_gather(0)
        for j in range(NR):
            phase(j, j)

        def body(g, carry):
            for p in range(NR):
                phase(NR + g * NR + p, p)
            return carry

        lax.fori_loop(0, (nch - NR) // NR, body, 0)
        wait_gather(nch % NR)
        wait_idx((nch + 1) % NR)
        wait_scatter((nch - 2) % NR)
        wait_scatter((nch - 1) % NR)
        plsc.subcore_barrier()
        pltpu.sync_copy(agg_sh.at[pl.ds(base_n, RPT)],
                        out_hbm.at[c, pl.ds(base_n, RPT)])

    return agg_kernel


BLK = 2048


def _layer1_body(dout_ref, din_ref, x_ref, w_ref, so_ref, si_ref, h_ref):
    deg_out = jnp.maximum(dout_ref[0, :] + dout_ref[1, :], 1.0)
    deg_in = jnp.maximum(din_ref[0, :] + din_ref[1, :], 1.0)
    so = lax.rsqrt(deg_out)
    si = lax.rsqrt(deg_in)
    so_ref[...] = so
    si_ref[...] = si
    h_ref[...] = jnp.dot(x_ref[...] * so[:, None], w_ref[...],
                         preferred_element_type=jnp.float32)


def _tc_layer1(dout_p, din_p, x_pad, W1):
    return pl.pallas_call(
        _layer1_body,
        grid=(NP // BLK,),
        in_specs=[
            pl.BlockSpec((NC, BLK), lambda i: (0, i)),
            pl.BlockSpec((NC, BLK), lambda i: (0, i)),
            pl.BlockSpec((BLK, D_IN), lambda i: (i, 0)),
            pl.BlockSpec((D_IN, D_H), lambda i: (0, 0)),
        ],
        out_specs=[
            pl.BlockSpec((BLK,), lambda i: (i,)),
            pl.BlockSpec((BLK,), lambda i: (i,)),
            pl.BlockSpec((BLK, D_H), lambda i: (i, 0)),
        ],
        out_shape=[
            jax.ShapeDtypeStruct((NP,), jnp.float32),
            jax.ShapeDtypeStruct((NP,), jnp.float32),
            jax.ShapeDtypeStruct((NP, D_H), jnp.float32),
        ],
    )(dout_p, din_p, x_pad, W1)


def _layer2_body(agg_ref, si_ref, so_ref, b_ref, w_ref, h_ref):
    agg = agg_ref[0, :, :] + agg_ref[1, :, :]
    h = agg * si_ref[...][:, None] + b_ref[...][None, :]
    h = jnp.maximum(h, 0.0)
    h_ref[...] = jnp.dot(h * so_ref[...][:, None], w_ref[...],
                         preferred_element_type=jnp.float32)


def _tc_layer2(aggp1, si, so, b1, W2):
    return pl.pallas_call(
        _layer2_body,
        grid=(NP // BLK,),
        in_specs=[
            pl.BlockSpec((NC, BLK, D_H), lambda i: (0, i, 0)),
            pl.BlockSpec((BLK,), lambda i: (i,)),
            pl.BlockSpec((BLK,), lambda i: (i,)),
            pl.BlockSpec((D_H,), lambda i: (0,)),
            pl.BlockSpec((D_H, D_OUT), lambda i: (0, 0)),
        ],
        out_specs=pl.BlockSpec((BLK, D_OUT), lambda i: (i, 0)),
        out_shape=jax.ShapeDtypeStruct((NP, D_OUT), jnp.float32),
    )(aggp1, si, so, b1, W2)


def _final_body(agg_ref, si_ref, b_ref, out_ref):
    agg = agg_ref[0, :, :] + agg_ref[1, :, :]
    out_ref[...] = agg * si_ref[...][:, None] + b_ref[...][None, :]


def _tc_final(aggp2, si, b2):
    return pl.pallas_call(
        _final_body,
        grid=(NP // BLK,),
        in_specs=[
            pl.BlockSpec((NC, BLK, D_OUT), lambda i: (0, i, 0)),
            pl.BlockSpec((BLK,), lambda i: (i,)),
            pl.BlockSpec((D_OUT,), lambda i: (0,)),
        ],
        out_specs=pl.BlockSpec((BLK, D_OUT), lambda i: (i, 0)),
        out_shape=jax.ShapeDtypeStruct((NP, D_OUT), jnp.float32),
    )(aggp2, si, b2)


def kernel(features, edge_index, W1, b1, W2, b2):
    pad_nodes = N + jax.lax.rem(jnp.arange(EP - E, dtype=jnp.int32),
                                jnp.int32(NP - N))
    epad = jnp.stack([pad_nodes, pad_nodes])
    ei = jnp.concatenate([edge_index, epad], axis=1)
    src = ei[0]
    dst = ei[1]
    x_pad = jnp.pad(features, ((0, NP - N), (0, 0)))
    zeros_n = jnp.zeros((NP,), jnp.float32)
    zeros_h = jnp.zeros((NP, D_H), jnp.float32)
    zeros_o = jnp.zeros((NP, D_OUT), jnp.float32)

    dout_p, din_p = _make_deg_kernel()(src, dst, zeros_n)
    so, si, h1 = _tc_layer1(dout_p, din_p, x_pad, W1)
    aggp1 = _make_agg_kernel(D_H, 80)(src, dst, h1, zeros_h)
    h2 = _tc_layer2(aggp1, si, so, b1, W2)
    aggp2 = _make_agg_kernel(D_OUT, 128)(src, dst, h2, zeros_o)
    return _tc_final(aggp2, si, b2)[:N]


# speedup vs baseline: 2.4201x; 1.0896x over previous
import functools

import jax
import jax.numpy as jnp
from jax import lax
from jax.experimental import pallas as pl
from jax.experimental.pallas import tpu as pltpu
from jax.experimental.pallas import tpu_sc as plsc

N = 10000
NP = 10240
E = 320000
EP = 327680
D_IN = 128
D_H = 128
D_OUT = 64

NC = 2
NS = 16
NW = NC * NS
EPW = EP // NW
RPT = NP // NS
NR = 4

NP2 = 10112
RPT2 = NP2 // NS
AGG_K = 128
AGG_NCH = EPW // AGG_K
NB = 3


def _sc_mesh():
    return plsc.VectorSubcoreMesh(core_axis_name="c", subcore_axis_name="s")


_SC_PARAMS = pltpu.CompilerParams(use_tc_tiling_on_sc=False)


DEG_K = 128
DEG_NCH = EPW // DEG_K


def _make_deg_kernel():
    @functools.partial(
        pl.kernel,
        out_type=(
            jax.ShapeDtypeStruct((NC, NP), jnp.float32),
            jax.ShapeDtypeStruct((NC, NP), jnp.float32),
        ),
        mesh=_sc_mesh(),
        scratch_types=(
            [pltpu.VMEM((DEG_K,), jnp.int32) for _ in range(2 * NR)]
            + [pltpu.VMEM((DEG_K,), jnp.float32),
               pltpu.VMEM_SHARED((NP,), jnp.float32),
               pltpu.VMEM_SHARED((NP,), jnp.float32)]
            + [pltpu.SemaphoreType.DMA for _ in range(NR + 1)]
        ),
        compiler_params=_SC_PARAMS,
    )
    def deg_kernel(src_hbm, dst_hbm, zeros_hbm, dout_hbm, din_hbm, *refs):
        sidx = refs[0:NR]
        didx = refs[NR:2 * NR]
        ones_v, dsrc_sh, ddst_sh = refs[2 * NR:2 * NR + 3]
        isem = refs[2 * NR + 3:2 * NR + 3 + NR]
        ssem = refs[2 * NR + 3 + NR]
        c = lax.axis_index("c")
        s = lax.axis_index("s")
        wid = c * NS + s
        base_n = s * RPT
        pltpu.sync_copy(zeros_hbm.at[pl.ds(base_n, RPT)],
                        dsrc_sh.at[pl.ds(base_n, RPT)])
        pltpu.sync_copy(zeros_hbm.at[pl.ds(base_n, RPT)],
                        ddst_sh.at[pl.ds(base_n, RPT)])
        for i in range(DEG_K // 16):
            ones_v[pl.ds(i * 16, 16)] = jnp.ones((16,), jnp.float32)
        plsc.subcore_barrier()

        def start_idx(j, m):
            base_e = wid * EPW + jnp.minimum(j, DEG_NCH - 1) * DEG_K
            pltpu.async_copy(src_hbm.at[pl.ds(base_e, DEG_K)], sidx[m],
                             isem[m])
            pltpu.async_copy(dst_hbm.at[pl.ds(base_e, DEG_K)], didx[m],
                             isem[m])

        def wait_idx(m):
            pltpu.make_async_copy(src_hbm.at[pl.ds(0, DEG_K)], sidx[m],
                                  isem[m]).wait()
            pltpu.make_async_copy(dst_hbm.at[pl.ds(0, DEG_K)], didx[m],
                                  isem[m]).wait()

        start_idx(0, 0)
        start_idx(1, 1)

        def body(g, carry):
            for p in range(NR):
                j = g * NR + p
                wait_idx(p)
                start_idx(j + 2, (p + 2) % NR)
                pltpu.async_copy(ones_v, dsrc_sh.at[sidx[p]], ssem, add=True)
                pltpu.sync_copy(ones_v, ddst_sh.at[didx[p]], add=True)
                pltpu.make_async_copy(zeros_hbm.at[pl.ds(0, DEG_K)], ones_v,
                                      ssem).wait()
            return carry

        lax.fori_loop(0, DEG_NCH // NR, body, 0)
        wait_idx(DEG_NCH % NR)
        wait_idx((DEG_NCH + 1) % NR)
        plsc.subcore_barrier()
        pltpu.sync_copy(dsrc_sh.at[pl.ds(base_n, RPT)],
                        dout_hbm.at[c, pl.ds(base_n, RPT)])
        pltpu.sync_copy(ddst_sh.at[pl.ds(base_n, RPT)],
                        din_hbm.at[c, pl.ds(base_n, RPT)])

    return deg_kernel


def _make_agg_kernel(d, k):
    nch = EPW // k
    assert nch % NR == 0

    @functools.partial(
        pl.kernel,
        out_type=jax.ShapeDtypeStruct((NC, NP, d), jnp.float32),
        mesh=_sc_mesh(),
        scratch_types=(
            [pltpu.VMEM((k,), jnp.int32) for _ in range(2 * NR)]
            + [pltpu.VMEM((k, d), jnp.float32) for _ in range(NR)]
            + [pltpu.VMEM_SHARED((NP, d), jnp.float32)]
            + [pltpu.SemaphoreType.DMA for _ in range(3 * NR)]
        ),
        compiler_params=_SC_PARAMS,
    )
    def agg_kernel(src_hbm, dst_hbm, h_hbm, zeros_hbm, out_hbm, *refs):
        sidx = refs[0:NR]
        didx = refs[NR:2 * NR]
        rows = refs[2 * NR:3 * NR]
        agg_sh = refs[3 * NR]
        isem = refs[3 * NR + 1:3 * NR + 1 + NR]
        gsem = refs[3 * NR + 1 + NR:3 * NR + 1 + 2 * NR]
        ssem = refs[3 * NR + 1 + 2 * NR:3 * NR + 1 + 3 * NR]
        c = lax.axis_index("c")
        s = lax.axis_index("s")
        wid = c * NS + s
        base_n = s * RPT
        pltpu.sync_copy(zeros_hbm.at[pl.ds(base_n, RPT)],
                        agg_sh.at[pl.ds(base_n, RPT)])

        def start_idx(j, m):
            base_e = wid * EPW + jnp.minimum(j, nch - 1) * k
            pltpu.async_copy(src_hbm.at[pl.ds(base_e, k)], sidx[m], isem[m])
            pltpu.async_copy(dst_hbm.at[pl.ds(base_e, k)], didx[m], isem[m])

        def wait_idx(m):
            pltpu.make_async_copy(src_hbm.at[pl.ds(0, k)], sidx[m],
                                  isem[m]).wait()
            pltpu.make_async_copy(dst_hbm.at[pl.ds(0, k)], didx[m],
                                  isem[m]).wait()

        def start_gather(m):
            pltpu.async_copy(h_hbm.at[sidx[m]], rows[m], gsem[m])

        def wait_gather(m):
            pltpu.make_async_copy(h_hbm.at[pl.ds(0, k)], rows[m],
                                  gsem[m]).wait()

        def start_scatter(m):
            pltpu.async_copy(rows[m], agg_sh.at[didx[m]], ssem[m], add=True)

        def wait_scatter(m):
            pltpu.make_async_copy(h_hbm.at[pl.ds(0, k)], rows[m],
                                  ssem[m]).wait()

        def phase(j, p):
            p1 = (p + 1) % NR
            p2 = (p + 2) % NR
            wait_gather(p)
            if not (isinstance(j, int) and j < 2):
                wait_scatter(p2)
            wait_idx(p1)
            start_gather(p1)
            start_idx(j + 2, p2)
            start_scatter(p)

        plsc.subcore_barrier()
        start_idx(0, 0)
        start_idx(1, 1)
        wait_idx(0)
        start_gather(0)
        for j in range(NR):
            phase(j, j)

        def body(g, carry):
            for p in range(NR):
                phase(NR + g * NR + p, p)
            return carry

        lax.fori_loop(0, (nch - NR) // NR, body, 0)
        wait_gather(nch % NR)
        wait_idx((nch + 1) % NR)
        wait_scatter((nch - 2) % NR)
        wait_scatter((nch - 1) % NR)
        plsc.subcore_barrier()
        pltpu.sync_copy(agg_sh.at[pl.ds(base_n, RPT)],
                        out_hbm.at[c, pl.ds(base_n, RPT)])

    return agg_kernel


BLK = 2048


def _layer1_body(dout_ref, din_ref, x_ref, w_ref, so_ref, si_ref, h_ref):
    deg_out = jnp.maximum(dout_ref[0, :] + dout_ref[1, :], 1.0)
    deg_in = jnp.maximum(din_ref[0, :] + din_ref[1, :], 1.0)
    so = lax.rsqrt(deg_out)
    si = lax.rsqrt(deg_in)
    so_ref[...] = so
    si_ref[...] = si
    h_ref[...] = jnp.dot(x_ref[...] * so[:, None], w_ref[...],
                         preferred_element_type=jnp.float32)


def _tc_layer1(dout_p, din_p, x_pad, W1):
    return pl.pallas_call(
        _layer1_body,
        grid=(NP // BLK,),
        in_specs=[
            pl.BlockSpec((NC, BLK), lambda i: (0, i)),
            pl.BlockSpec((NC, BLK), lambda i: (0, i)),
            pl.BlockSpec((BLK, D_IN), lambda i: (i, 0)),
            pl.BlockSpec((D_IN, D_H), lambda i: (0, 0)),
        ],
        out_specs=[
            pl.BlockSpec((BLK,), lambda i: (i,)),
            pl.BlockSpec((BLK,), lambda i: (i,)),
            pl.BlockSpec((BLK, D_H), lambda i: (i, 0)),
        ],
        out_shape=[
            jax.ShapeDtypeStruct((NP,), jnp.float32),
            jax.ShapeDtypeStruct((NP,), jnp.float32),
            jax.ShapeDtypeStruct((NP, D_H), jnp.float32),
        ],
    )(dout_p, din_p, x_pad, W1)


def _layer2_body(agg_ref, si_ref, so_ref, b_ref, w_ref, h_ref):
    agg = agg_ref[0, :, :] + agg_ref[1, :, :]
    h = agg * si_ref[...][:, None] + b_ref[...][None, :]
    h = jnp.maximum(h, 0.0)
    h_ref[...] = jnp.dot(h * so_ref[...][:, None], w_ref[...],
                         preferred_element_type=jnp.float32)


def _tc_layer2(aggp1, si, so, b1, W2):
    return pl.pallas_call(
        _layer2_body,
        grid=(NP // BLK,),
        in_specs=[
            pl.BlockSpec((NC, BLK, D_H), lambda i: (0, i, 0)),
            pl.BlockSpec((BLK,), lambda i: (i,)),
            pl.BlockSpec((BLK,), lambda i: (i,)),
            pl.BlockSpec((D_H,), lambda i: (0,)),
            pl.BlockSpec((D_H, D_OUT), lambda i: (0, 0)),
        ],
        out_specs=pl.BlockSpec((BLK, D_OUT), lambda i: (i, 0)),
        out_shape=jax.ShapeDtypeStruct((NP, D_OUT), jnp.float32),
    )(aggp1, si, so, b1, W2)


def _final_body(agg_ref, si_ref, b_ref, out_ref):
    agg = agg_ref[0, :, :] + agg_ref[1, :, :]
    out_ref[...] = agg * si_ref[...][:, None] + b_ref[...][None, :]


def _tc_final(aggp2, si, b2):
    return pl.pallas_call(
        _final_body,
        grid=(NP // BLK,),
        in_specs=[
            pl.BlockSpec((NC, BLK, D_OUT), lambda i: (0, i, 0)),
            pl.BlockSpec((BLK,), lambda i: (i,)),
            pl.BlockSpec((D_OUT,), lambda i: (0,)),
        ],
        out_specs=pl.BlockSpec((BLK, D_OUT), lambda i: (i, 0)),
        out_shape=jax.ShapeDtypeStruct((NP, D_OUT), jnp.float32),
    )(aggp2, si, b2)


def kernel(features, edge_index, W1, b1, W2, b2):
    pad_nodes = N + jax.lax.rem(jnp.arange(EP - E, dtype=jnp.int32),
                                jnp.int32(NP - N))
    epad = jnp.stack([pad_nodes, pad_nodes])
    ei = jnp.concatenate([edge_index, epad], axis=1)
    src = ei[0]
    dst = ei[1]
    x_pad = jnp.pad(features, ((0, NP - N), (0, 0)))
    zeros_n = jnp.zeros((NP,), jnp.float32)
    zeros_h = jnp.zeros((NP, D_H), jnp.float32)
    zeros_o = jnp.zeros((NP, D_OUT), jnp.float32)

    dout_p, din_p = _make_deg_kernel()(src, dst, zeros_n)
    so, si, h1 = _tc_layer1(dout_p, din_p, x_pad, W1)
    aggp1 = _make_agg_kernel(D_H, 80)(src, dst, h1, zeros_h)
    h2 = _tc_layer2(aggp1, si, so, b1, W2)
    aggp2 = _make_agg_kernel(D_OUT, 128)(src, dst, h2, zeros_o)
    return _tc_final(aggp2, si, b2)[:N]


# speedup vs baseline: 2.6635x; 1.1006x over previous
import functools

import jax
import jax.numpy as jnp
from jax import lax
from jax.experimental import pallas as pl
from jax.experimental.pallas import tpu as pltpu
from jax.experimental.pallas import tpu_sc as plsc

N = 10000
NP = 10240
E = 320000
EP = 327680
D_IN = 128
D_H = 128
D_OUT = 64

NC = 2
NS = 16
NW = NC * NS
EPW = EP // NW
RPT = NP // NS
NR = 4

NP2 = 10112
RPT2 = NP2 // NS
AGG_K = 128
AGG_NCH = EPW // AGG_K
NB = 3


def _sc_mesh():
    return plsc.VectorSubcoreMesh(core_axis_name="c", subcore_axis_name="s")


_SC_PARAMS = pltpu.CompilerParams(use_tc_tiling_on_sc=False)


DEG_K = 128
DEG_NCH = EPW // DEG_K


def _make_deg_kernel():
    @functools.partial(
        pl.kernel,
        out_type=(
            jax.ShapeDtypeStruct((NC, NP), jnp.float32),
            jax.ShapeDtypeStruct((NC, NP), jnp.float32),
        ),
        mesh=_sc_mesh(),
        scratch_types=(
            [pltpu.VMEM((DEG_K,), jnp.int32) for _ in range(2 * NR)]
            + [pltpu.VMEM((DEG_K,), jnp.float32),
               pltpu.VMEM_SHARED((NP,), jnp.float32),
               pltpu.VMEM_SHARED((NP,), jnp.float32)]
            + [pltpu.SemaphoreType.DMA for _ in range(NR + 1)]
        ),
        compiler_params=_SC_PARAMS,
    )
    def deg_kernel(src_hbm, dst_hbm, zeros_hbm, dout_hbm, din_hbm, *refs):
        sidx = refs[0:NR]
        didx = refs[NR:2 * NR]
        ones_v, dsrc_sh, ddst_sh = refs[2 * NR:2 * NR + 3]
        isem = refs[2 * NR + 3:2 * NR + 3 + NR]
        ssem = refs[2 * NR + 3 + NR]
        c = lax.axis_index("c")
        s = lax.axis_index("s")
        wid = c * NS + s
        base_n = s * RPT
        pltpu.sync_copy(zeros_hbm.at[pl.ds(base_n, RPT)],
                        dsrc_sh.at[pl.ds(base_n, RPT)])
        pltpu.sync_copy(zeros_hbm.at[pl.ds(base_n, RPT)],
                        ddst_sh.at[pl.ds(base_n, RPT)])
        for i in range(DEG_K // 16):
            ones_v[pl.ds(i * 16, 16)] = jnp.ones((16,), jnp.float32)
        plsc.subcore_barrier()

        def start_idx(j, m):
            base_e = wid * EPW + jnp.minimum(j, DEG_NCH - 1) * DEG_K
            pltpu.async_copy(src_hbm.at[pl.ds(base_e, DEG_K)], sidx[m],
                             isem[m])
            pltpu.async_copy(dst_hbm.at[pl.ds(base_e, DEG_K)], didx[m],
                             isem[m])

        def wait_idx(m):
            pltpu.make_async_copy(src_hbm.at[pl.ds(0, DEG_K)], sidx[m],
                                  isem[m]).wait()
            pltpu.make_async_copy(dst_hbm.at[pl.ds(0, DEG_K)], didx[m],
                                  isem[m]).wait()

        start_idx(0, 0)
        start_idx(1, 1)

        def body(g, carry):
            for p in range(NR):
                j = g * NR + p
                wait_idx(p)
                start_idx(j + 2, (p + 2) % NR)
                pltpu.async_copy(ones_v, dsrc_sh.at[sidx[p]], ssem, add=True)
                pltpu.sync_copy(ones_v, ddst_sh.at[didx[p]], add=True)
                pltpu.make_async_copy(zeros_hbm.at[pl.ds(0, DEG_K)], ones_v,
                                      ssem).wait()
            return carry

        lax.fori_loop(0, DEG_NCH // NR, body, 0)
        wait_idx(DEG_NCH % NR)
        wait_idx((DEG_NCH + 1) % NR)
        plsc.subcore_barrier()
        pltpu.sync_copy(dsrc_sh.at[pl.ds(base_n, RPT)],
                        dout_hbm.at[c, pl.ds(base_n, RPT)])
        pltpu.sync_copy(ddst_sh.at[pl.ds(base_n, RPT)],
                        din_hbm.at[c, pl.ds(base_n, RPT)])

    return deg_kernel


def _make_agg_kernel(d, k):
    nch = EPW // k
    assert nch % NR == 0

    @functools.partial(
        pl.kernel,
        out_type=jax.ShapeDtypeStruct((NC, NP, d), jnp.bfloat16),
        mesh=_sc_mesh(),
        scratch_types=(
            [pltpu.VMEM((k,), jnp.int32) for _ in range(2 * NR)]
            + [pltpu.VMEM((k, d), jnp.bfloat16) for _ in range(NR)]
            + [pltpu.VMEM_SHARED((NP, d), jnp.bfloat16)]
            + [pltpu.SemaphoreType.DMA for _ in range(3 * NR)]
        ),
        compiler_params=_SC_PARAMS,
    )
    def agg_kernel(src_hbm, dst_hbm, h_hbm, zeros_hbm, out_hbm, *refs):
        sidx = refs[0:NR]
        didx = refs[NR:2 * NR]
        rows = refs[2 * NR:3 * NR]
        agg_sh = refs[3 * NR]
        isem = refs[3 * NR + 1:3 * NR + 1 + NR]
        gsem = refs[3 * NR + 1 + NR:3 * NR + 1 + 2 * NR]
        ssem = refs[3 * NR + 1 + 2 * NR:3 * NR + 1 + 3 * NR]
        c = lax.axis_index("c")
        s = lax.axis_index("s")
        wid = c * NS + s
        base_n = s * RPT
        pltpu.sync_copy(zeros_hbm.at[pl.ds(base_n, RPT)],
                        agg_sh.at[pl.ds(base_n, RPT)])

        def start_idx(j, m):
            base_e = wid * EPW + jnp.minimum(j, nch - 1) * k
            pltpu.async_copy(src_hbm.at[pl.ds(base_e, k)], sidx[m], isem[m])
            pltpu.async_copy(dst_hbm.at[pl.ds(base_e, k)], didx[m], isem[m])

        def wait_idx(m):
            pltpu.make_async_copy(src_hbm.at[pl.ds(0, k)], sidx[m],
                                  isem[m]).wait()
            pltpu.make_async_copy(dst_hbm.at[pl.ds(0, k)], didx[m],
                                  isem[m]).wait()

        def start_gather(m):
            pltpu.async_copy(h_hbm.at[sidx[m]], rows[m], gsem[m])

        def wait_gather(m):
            pltpu.make_async_copy(h_hbm.at[pl.ds(0, k)], rows[m],
                                  gsem[m]).wait()

        def start_scatter(m):
            pltpu.async_copy(rows[m], agg_sh.at[didx[m]], ssem[m], add=True)

        def wait_scatter(m):
            pltpu.make_async_copy(h_hbm.at[pl.ds(0, k)], rows[m],
                                  ssem[m]).wait()

        def phase(j, p):
            p1 = (p + 1) % NR
            p2 = (p + 2) % NR
            wait_gather(p)
            if not (isinstance(j, int) and j < 2):
                wait_scatter(p2)
            wait_idx(p1)
            start_gather(p1)
            start_idx(j + 2, p2)
            start_scatter(p)

        plsc.subcore_barrier()
        start_idx(0, 0)
        start_idx(1, 1)
        wait_idx(0)
        start_gather(0)
        for j in range(NR):
            phase(j, j)

        def body(g, carry):
            for p in range(NR):
                phase(NR + g * NR + p, p)
            return carry

        lax.fori_loop(0, (nch - NR) // NR, body, 0)
        wait_gather(nch % NR)
        wait_idx((nch + 1) % NR)
        wait_scatter((nch - 2) % NR)
        wait_scatter((nch - 1) % NR)
        plsc.subcore_barrier()
        pltpu.sync_copy(agg_sh.at[pl.ds(base_n, RPT)],
                        out_hbm.at[c, pl.ds(base_n, RPT)])

    return agg_kernel


BLK = 2048


def _layer1_body(dout_ref, din_ref, x_ref, w_ref, so_ref, si_ref, h_ref):
    deg_out = jnp.maximum(dout_ref[0, :] + dout_ref[1, :], 1.0)
    deg_in = jnp.maximum(din_ref[0, :] + din_ref[1, :], 1.0)
    so = lax.rsqrt(deg_out)
    si = lax.rsqrt(deg_in)
    so_ref[...] = so
    si_ref[...] = si
    h_ref[...] = jnp.dot(x_ref[...] * so[:, None], w_ref[...],
                         preferred_element_type=jnp.float32
                         ).astype(jnp.bfloat16)


def _tc_layer1(dout_p, din_p, x_pad, W1):
    return pl.pallas_call(
        _layer1_body,
        grid=(NP // BLK,),
        in_specs=[
            pl.BlockSpec((NC, BLK), lambda i: (0, i)),
            pl.BlockSpec((NC, BLK), lambda i: (0, i)),
            pl.BlockSpec((BLK, D_IN), lambda i: (i, 0)),
            pl.BlockSpec((D_IN, D_H), lambda i: (0, 0)),
        ],
        out_specs=[
            pl.BlockSpec((BLK,), lambda i: (i,)),
            pl.BlockSpec((BLK,), lambda i: (i,)),
            pl.BlockSpec((BLK, D_H), lambda i: (i, 0)),
        ],
        out_shape=[
            jax.ShapeDtypeStruct((NP,), jnp.float32),
            jax.ShapeDtypeStruct((NP,), jnp.float32),
            jax.ShapeDtypeStruct((NP, D_H), jnp.bfloat16),
        ],
    )(dout_p, din_p, x_pad, W1)


def _layer2_body(agg_ref, si_ref, so_ref, b_ref, w_ref, h_ref):
    agg = (agg_ref[0, :, :].astype(jnp.float32)
           + agg_ref[1, :, :].astype(jnp.float32))
    h = agg * si_ref[...][:, None] + b_ref[...][None, :]
    h = jnp.maximum(h, 0.0)
    h_ref[...] = jnp.dot(h * so_ref[...][:, None], w_ref[...],
                         preferred_element_type=jnp.float32
                         ).astype(jnp.bfloat16)


def _tc_layer2(aggp1, si, so, b1, W2):
    return pl.pallas_call(
        _layer2_body,
        grid=(NP // BLK,),
        in_specs=[
            pl.BlockSpec((NC, BLK, D_H), lambda i: (0, i, 0)),
            pl.BlockSpec((BLK,), lambda i: (i,)),
            pl.BlockSpec((BLK,), lambda i: (i,)),
            pl.BlockSpec((D_H,), lambda i: (0,)),
            pl.BlockSpec((D_H, D_OUT), lambda i: (0, 0)),
        ],
        out_specs=pl.BlockSpec((BLK, D_OUT), lambda i: (i, 0)),
        out_shape=jax.ShapeDtypeStruct((NP, D_OUT), jnp.bfloat16),
    )(aggp1, si, so, b1, W2)


def _final_body(agg_ref, si_ref, b_ref, out_ref):
    agg = (agg_ref[0, :, :].astype(jnp.float32)
           + agg_ref[1, :, :].astype(jnp.float32))
    out_ref[...] = agg * si_ref[...][:, None] + b_ref[...][None, :]


def _tc_final(aggp2, si, b2):
    return pl.pallas_call(
        _final_body,
        grid=(NP // BLK,),
        in_specs=[
            pl.BlockSpec((NC, BLK, D_OUT), lambda i: (0, i, 0)),
            pl.BlockSpec((BLK,), lambda i: (i,)),
            pl.BlockSpec((D_OUT,), lambda i: (0,)),
        ],
        out_specs=pl.BlockSpec((BLK, D_OUT), lambda i: (i, 0)),
        out_shape=jax.ShapeDtypeStruct((NP, D_OUT), jnp.float32),
    )(aggp2, si, b2)


def kernel(features, edge_index, W1, b1, W2, b2):
    pad_nodes = N + jax.lax.rem(jnp.arange(EP - E, dtype=jnp.int32),
                                jnp.int32(NP - N))
    epad = jnp.stack([pad_nodes, pad_nodes])
    ei = jnp.concatenate([edge_index, epad], axis=1)
    src = ei[0]
    dst = ei[1]
    x_pad = jnp.pad(features, ((0, NP - N), (0, 0)))
    zeros_n = jnp.zeros((NP,), jnp.float32)
    zeros_h = jnp.zeros((NP, D_H), jnp.bfloat16)
    zeros_o = jnp.zeros((NP, D_OUT), jnp.bfloat16)

    dout_p, din_p = _make_deg_kernel()(src, dst, zeros_n)
    so, si, h1 = _tc_layer1(dout_p, din_p, x_pad, W1)
    aggp1 = _make_agg_kernel(D_H, 128)(src, dst, h1, zeros_h)
    h2 = _tc_layer2(aggp1, si, so, b1, W2)
    aggp2 = _make_agg_kernel(D_OUT, 128)(src, dst, h2, zeros_o)
    return _tc_final(aggp2, si, b2)[:N]


# speedup vs baseline: 3.1323x; 1.1760x over previous
import functools

import jax
import jax.numpy as jnp
from jax import lax
from jax.experimental import pallas as pl
from jax.experimental.pallas import tpu as pltpu
from jax.experimental.pallas import tpu_sc as plsc

N = 10000
NP = 10240
E = 320000
EP = 327680
D_IN = 128
D_H = 128
D_OUT = 64

NC = 2
NS = 16
NW = NC * NS
EPW = EP // NW
RPT = NP // NS
NR = 4

NP2 = 10112
RPT2 = NP2 // NS
AGG_K = 128
AGG_NCH = EPW // AGG_K
NB = 3


def _sc_mesh():
    return plsc.VectorSubcoreMesh(core_axis_name="c", subcore_axis_name="s")


_SC_PARAMS = pltpu.CompilerParams(use_tc_tiling_on_sc=False)


DEG_K = 128
DEG_NCH = EPW // DEG_K


def _make_deg_kernel():
    @functools.partial(
        pl.kernel,
        out_type=(
            jax.ShapeDtypeStruct((NC, NP), jnp.float32),
            jax.ShapeDtypeStruct((NC, NP), jnp.float32),
        ),
        mesh=_sc_mesh(),
        scratch_types=(
            [pltpu.VMEM((DEG_K,), jnp.int32) for _ in range(2 * NR)]
            + [pltpu.VMEM((DEG_K,), jnp.float32),
               pltpu.VMEM_SHARED((NP,), jnp.float32),
               pltpu.VMEM_SHARED((NP,), jnp.float32)]
            + [pltpu.SemaphoreType.DMA for _ in range(NR + 1)]
        ),
        compiler_params=_SC_PARAMS,
    )
    def deg_kernel(src_hbm, dst_hbm, zeros_hbm, dout_hbm, din_hbm, *refs):
        sidx = refs[0:NR]
        didx = refs[NR:2 * NR]
        ones_v, dsrc_sh, ddst_sh = refs[2 * NR:2 * NR + 3]
        isem = refs[2 * NR + 3:2 * NR + 3 + NR]
        ssem = refs[2 * NR + 3 + NR]
        c = lax.axis_index("c")
        s = lax.axis_index("s")
        wid = c * NS + s
        base_n = s * RPT
        pltpu.sync_copy(zeros_hbm.at[pl.ds(base_n, RPT)],
                        dsrc_sh.at[pl.ds(base_n, RPT)])
        pltpu.sync_copy(zeros_hbm.at[pl.ds(base_n, RPT)],
                        ddst_sh.at[pl.ds(base_n, RPT)])
        for i in range(DEG_K // 16):
            ones_v[pl.ds(i * 16, 16)] = jnp.ones((16,), jnp.float32)
        plsc.subcore_barrier()

        def start_idx(j, m):
            base_e = wid * EPW + jnp.minimum(j, DEG_NCH - 1) * DEG_K
            pltpu.async_copy(src_hbm.at[pl.ds(base_e, DEG_K)], sidx[m],
                             isem[m])
            pltpu.async_copy(dst_hbm.at[pl.ds(base_e, DEG_K)], didx[m],
                             isem[m])

        def wait_idx(m):
            pltpu.make_async_copy(src_hbm.at[pl.ds(0, DEG_K)], sidx[m],
                                  isem[m]).wait()
            pltpu.make_async_copy(dst_hbm.at[pl.ds(0, DEG_K)], didx[m],
                                  isem[m]).wait()

        start_idx(0, 0)
        start_idx(1, 1)

        def body(g, carry):
            for p in range(NR):
                j = g * NR + p
                wait_idx(p)
                start_idx(j + 2, (p + 2) % NR)
                pltpu.async_copy(ones_v, dsrc_sh.at[sidx[p]], ssem, add=True)
                pltpu.sync_copy(ones_v, ddst_sh.at[didx[p]], add=True)
                pltpu.make_async_copy(zeros_hbm.at[pl.ds(0, DEG_K)], ones_v,
                                      ssem).wait()
            return carry

        lax.fori_loop(0, DEG_NCH // NR, body, 0)
        wait_idx(DEG_NCH % NR)
        wait_idx((DEG_NCH + 1) % NR)
        plsc.subcore_barrier()
        pltpu.sync_copy(dsrc_sh.at[pl.ds(base_n, RPT)],
                        dout_hbm.at[c, pl.ds(base_n, RPT)])
        pltpu.sync_copy(ddst_sh.at[pl.ds(base_n, RPT)],
                        din_hbm.at[c, pl.ds(base_n, RPT)])

    return deg_kernel


def _make_agg_kernel(d, k):
    nch = EPW // k
    assert nch % NR == 0

    @functools.partial(
        pl.kernel,
        out_type=jax.ShapeDtypeStruct((NC, NP, d), jnp.bfloat16),
        mesh=_sc_mesh(),
        scratch_types=(
            [pltpu.VMEM((k,), jnp.int32) for _ in range(2 * NR)]
            + [pltpu.VMEM((k, d), jnp.bfloat16) for _ in range(NR)]
            + [pltpu.VMEM_SHARED((NP, d), jnp.bfloat16)]
            + [pltpu.SemaphoreType.DMA for _ in range(3 * NR)]
        ),
        compiler_params=_SC_PARAMS,
    )
    def agg_kernel(src_hbm, dst_hbm, h_hbm, zeros_hbm, out_hbm, *refs):
        sidx = refs[0:NR]
        didx = refs[NR:2 * NR]
        rows = refs[2 * NR:3 * NR]
        agg_sh = refs[3 * NR]
        isem = refs[3 * NR + 1:3 * NR + 1 + NR]
        gsem = refs[3 * NR + 1 + NR:3 * NR + 1 + 2 * NR]
        ssem = refs[3 * NR + 1 + 2 * NR:3 * NR + 1 + 3 * NR]
        c = lax.axis_index("c")
        s = lax.axis_index("s")
        wid = c * NS + s
        base_n = s * RPT
        pltpu.sync_copy(zeros_hbm.at[pl.ds(base_n, RPT)],
                        agg_sh.at[pl.ds(base_n, RPT)])

        def start_idx(j, m):
            base_e = wid * EPW + jnp.minimum(j, nch - 1) * k
            pltpu.async_copy(src_hbm.at[pl.ds(base_e, k)], sidx[m], isem[m])
            pltpu.async_copy(dst_hbm.at[pl.ds(base_e, k)], didx[m], isem[m])

        def wait_idx(m):
            pltpu.make_async_copy(src_hbm.at[pl.ds(0, k)], sidx[m],
                                  isem[m]).wait()
            pltpu.make_async_copy(dst_hbm.at[pl.ds(0, k)], didx[m],
                                  isem[m]).wait()

        def start_gather(m):
            pltpu.async_copy(h_hbm.at[sidx[m]], rows[m], gsem[m])

        def wait_gather(m):
            pltpu.make_async_copy(h_hbm.at[pl.ds(0, k)], rows[m],
                                  gsem[m]).wait()

        def start_scatter(m):
            pltpu.async_copy(rows[m], agg_sh.at[didx[m]], ssem[m], add=True)

        def wait_scatter(m):
            pltpu.make_async_copy(h_hbm.at[pl.ds(0, k)], rows[m],
                                  ssem[m]).wait()

        def phase(j, p):
            p1 = (p + 1) % NR
            p2 = (p + 2) % NR
            wait_gather(p)
            if not (isinstance(j, int) and j < 2):
                wait_scatter(p2)
            wait_idx(p1)
            start_gather(p1)
            start_idx(j + 2, p2)
            start_scatter(p)

        plsc.subcore_barrier()
        start_idx(0, 0)
        start_idx(1, 1)
        wait_idx(0)
        start_gather(0)
        for j in range(NR):
            phase(j, j)

        def body(g, carry):
            for p in range(NR):
                phase(NR + g * NR + p, p)
            return carry

        lax.fori_loop(0, (nch - NR) // NR, body, 0)
        wait_gather(nch % NR)
        wait_idx((nch + 1) % NR)
        wait_scatter((nch - 2) % NR)
        wait_scatter((nch - 1) % NR)
        plsc.subcore_barrier()
        pltpu.sync_copy(agg_sh.at[pl.ds(base_n, RPT)],
                        out_hbm.at[c, pl.ds(base_n, RPT)])

    return agg_kernel


BLK = 2048


def _xw_body(x_ref, w_ref, xw_ref):
    xw_ref[...] = jnp.dot(x_ref[...], w_ref[...],
                          preferred_element_type=jnp.float32)


def _tc_xw(x_pad, W1):
    return pl.pallas_call(
        _xw_body,
        grid=(NP // BLK,),
        in_specs=[
            pl.BlockSpec((BLK, D_IN), lambda i: (i, 0)),
            pl.BlockSpec((D_IN, D_H), lambda i: (0, 0)),
        ],
        out_specs=pl.BlockSpec((BLK, D_H), lambda i: (i, 0)),
        out_shape=jax.ShapeDtypeStruct((NP, D_H), jnp.float32),
    )(x_pad, W1)


def _layer1_body(dout_ref, din_ref, xw_ref, so_ref, si_ref, h_ref):
    deg_out = jnp.maximum(dout_ref[0, :] + dout_ref[1, :], 1.0)
    deg_in = jnp.maximum(din_ref[0, :] + din_ref[1, :], 1.0)
    so = lax.rsqrt(deg_out)
    si = lax.rsqrt(deg_in)
    so_ref[...] = so
    si_ref[...] = si
    h_ref[...] = (xw_ref[...] * so[:, None]).astype(jnp.bfloat16)


def _tc_layer1(dout_p, din_p, xw, W1):
    return pl.pallas_call(
        _layer1_body,
        grid=(NP // BLK,),
        in_specs=[
            pl.BlockSpec((NC, BLK), lambda i: (0, i)),
            pl.BlockSpec((NC, BLK), lambda i: (0, i)),
            pl.BlockSpec((BLK, D_H), lambda i: (i, 0)),
        ],
        out_specs=[
            pl.BlockSpec((BLK,), lambda i: (i,)),
            pl.BlockSpec((BLK,), lambda i: (i,)),
            pl.BlockSpec((BLK, D_H), lambda i: (i, 0)),
        ],
        out_shape=[
            jax.ShapeDtypeStruct((NP,), jnp.float32),
            jax.ShapeDtypeStruct((NP,), jnp.float32),
            jax.ShapeDtypeStruct((NP, D_H), jnp.bfloat16),
        ],
    )(dout_p, din_p, xw)


def _layer2_body(agg_ref, si_ref, so_ref, b_ref, w_ref, h_ref):
    agg = (agg_ref[0, :, :].astype(jnp.float32)
           + agg_ref[1, :, :].astype(jnp.float32))
    h = agg * si_ref[...][:, None] + b_ref[...][None, :]
    h = jnp.maximum(h, 0.0)
    h_ref[...] = jnp.dot(h * so_ref[...][:, None], w_ref[...],
                         preferred_element_type=jnp.float32
                         ).astype(jnp.bfloat16)


def _tc_layer2(aggp1, si, so, b1, W2):
    return pl.pallas_call(
        _layer2_body,
        grid=(NP // BLK,),
        in_specs=[
            pl.BlockSpec((NC, BLK, D_H), lambda i: (0, i, 0)),
            pl.BlockSpec((BLK,), lambda i: (i,)),
            pl.BlockSpec((BLK,), lambda i: (i,)),
            pl.BlockSpec((D_H,), lambda i: (0,)),
            pl.BlockSpec((D_H, D_OUT), lambda i: (0, 0)),
        ],
        out_specs=pl.BlockSpec((BLK, D_OUT), lambda i: (i, 0)),
        out_shape=jax.ShapeDtypeStruct((NP, D_OUT), jnp.bfloat16),
    )(aggp1, si, so, b1, W2)


def _final_body(agg_ref, si_ref, b_ref, out_ref):
    agg = (agg_ref[0, :, :].astype(jnp.float32)
           + agg_ref[1, :, :].astype(jnp.float32))
    out_ref[...] = agg * si_ref[...][:, None] + b_ref[...][None, :]


def _tc_final(aggp2, si, b2):
    return pl.pallas_call(
        _final_body,
        grid=(NP // BLK,),
        in_specs=[
            pl.BlockSpec((NC, BLK, D_OUT), lambda i: (0, i, 0)),
            pl.BlockSpec((BLK,), lambda i: (i,)),
            pl.BlockSpec((D_OUT,), lambda i: (0,)),
        ],
        out_specs=pl.BlockSpec((BLK, D_OUT), lambda i: (i, 0)),
        out_shape=jax.ShapeDtypeStruct((NP, D_OUT), jnp.float32),
    )(aggp2, si, b2)


def kernel(features, edge_index, W1, b1, W2, b2):
    pad_nodes = N + jax.lax.rem(jnp.arange(EP - E, dtype=jnp.int32),
                                jnp.int32(NP - N))
    epad = jnp.stack([pad_nodes, pad_nodes])
    ei = jnp.concatenate([edge_index, epad], axis=1)
    src = ei[0]
    dst = ei[1]
    x_pad = jnp.pad(features, ((0, NP - N), (0, 0)))
    zeros_n = jnp.zeros((NP,), jnp.float32)
    zeros_h = jnp.zeros((NP, D_H), jnp.bfloat16)
    zeros_o = jnp.zeros((NP, D_OUT), jnp.bfloat16)

    xw = _tc_xw(x_pad, W1)
    dout_p, din_p = _make_deg_kernel()(src, dst, zeros_n)
    so, si, h1 = _tc_layer1(dout_p, din_p, xw, W1)
    aggp1 = _make_agg_kernel(D_H, 256)(src, dst, h1, zeros_h)
    h2 = _tc_layer2(aggp1, si, so, b1, W2)
    aggp2 = _make_agg_kernel(D_OUT, 256)(src, dst, h2, zeros_o)
    return _tc_final(aggp2, si, b2)[:N]


# speedup vs baseline: 3.3830x; 1.0801x over previous
import functools

import jax
import jax.numpy as jnp
from jax import lax
from jax.experimental import pallas as pl
from jax.experimental.pallas import tpu as pltpu
from jax.experimental.pallas import tpu_sc as plsc

N = 10000
NP = 10240
E = 320000
EP = 327680
D_IN = 128
D_H = 128
D_OUT = 64

NC = 2
NS = 16
NW = NC * NS
EPW = EP // NW
RPT = NP // NS
NR = 4

NP2 = 10112
RPT2 = NP2 // NS
AGG_K = 128
AGG_NCH = EPW // AGG_K
NB = 3


def _sc_mesh():
    return plsc.VectorSubcoreMesh(core_axis_name="c", subcore_axis_name="s")


_SC_PARAMS = pltpu.CompilerParams(use_tc_tiling_on_sc=False)


DEG_K = 256
DEG_NCH = EPW // DEG_K


def _make_deg_kernel():
    @functools.partial(
        pl.kernel,
        out_type=(
            jax.ShapeDtypeStruct((NC, NP), jnp.float32),
            jax.ShapeDtypeStruct((NC, NP), jnp.float32),
        ),
        mesh=_sc_mesh(),
        scratch_types=(
            [pltpu.VMEM((DEG_K,), jnp.int32) for _ in range(2 * NR)]
            + [pltpu.VMEM((DEG_K,), jnp.float32),
               pltpu.VMEM_SHARED((NP,), jnp.float32),
               pltpu.VMEM_SHARED((NP,), jnp.float32)]
            + [pltpu.SemaphoreType.DMA for _ in range(NR + 1)]
        ),
        compiler_params=_SC_PARAMS,
    )
    def deg_kernel(src_hbm, dst_hbm, zeros_hbm, dout_hbm, din_hbm, *refs):
        sidx = refs[0:NR]
        didx = refs[NR:2 * NR]
        ones_v, dsrc_sh, ddst_sh = refs[2 * NR:2 * NR + 3]
        isem = refs[2 * NR + 3:2 * NR + 3 + NR]
        ssem = refs[2 * NR + 3 + NR]
        c = lax.axis_index("c")
        s = lax.axis_index("s")
        wid = c * NS + s
        base_n = s * RPT
        pltpu.sync_copy(zeros_hbm.at[pl.ds(base_n, RPT)],
                        dsrc_sh.at[pl.ds(base_n, RPT)])
        pltpu.sync_copy(zeros_hbm.at[pl.ds(base_n, RPT)],
                        ddst_sh.at[pl.ds(base_n, RPT)])
        for i in range(DEG_K // 16):
            ones_v[pl.ds(i * 16, 16)] = jnp.ones((16,), jnp.float32)
        plsc.subcore_barrier()

        def start_idx(j, m):
            base_e = wid * EPW + jnp.minimum(j, DEG_NCH - 1) * DEG_K
            pltpu.async_copy(src_hbm.at[pl.ds(base_e, DEG_K)], sidx[m],
                             isem[m])
            pltpu.async_copy(dst_hbm.at[pl.ds(base_e, DEG_K)], didx[m],
                             isem[m])

        def wait_idx(m):
            pltpu.make_async_copy(src_hbm.at[pl.ds(0, DEG_K)], sidx[m],
                                  isem[m]).wait()
            pltpu.make_async_copy(dst_hbm.at[pl.ds(0, DEG_K)], didx[m],
                                  isem[m]).wait()

        start_idx(0, 0)
        start_idx(1, 1)

        def body(g, carry):
            for p in range(NR):
                j = g * NR + p
                wait_idx(p)
                start_idx(j + 2, (p + 2) % NR)
                pltpu.async_copy(ones_v, dsrc_sh.at[sidx[p]], ssem, add=True)
                pltpu.sync_copy(ones_v, ddst_sh.at[didx[p]], add=True)
                pltpu.make_async_copy(zeros_hbm.at[pl.ds(0, DEG_K)], ones_v,
                                      ssem).wait()
            return carry

        lax.fori_loop(0, DEG_NCH // NR, body, 0)
        wait_idx(DEG_NCH % NR)
        wait_idx((DEG_NCH + 1) % NR)
        plsc.subcore_barrier()
        pltpu.sync_copy(dsrc_sh.at[pl.ds(base_n, RPT)],
                        dout_hbm.at[c, pl.ds(base_n, RPT)])
        pltpu.sync_copy(ddst_sh.at[pl.ds(base_n, RPT)],
                        din_hbm.at[c, pl.ds(base_n, RPT)])

    return deg_kernel


def _make_agg_kernel(d, k):
    nch = EPW // k
    assert nch % NR == 0

    @functools.partial(
        pl.kernel,
        out_type=jax.ShapeDtypeStruct((NC, NP, d), jnp.bfloat16),
        mesh=_sc_mesh(),
        scratch_types=(
            [pltpu.VMEM((k,), jnp.int32) for _ in range(2 * NR)]
            + [pltpu.VMEM((k, d), jnp.bfloat16) for _ in range(NR)]
            + [pltpu.VMEM_SHARED((NP, d), jnp.bfloat16)]
            + [pltpu.SemaphoreType.DMA for _ in range(3 * NR)]
        ),
        compiler_params=_SC_PARAMS,
    )
    def agg_kernel(src_hbm, dst_hbm, h_hbm, zeros_hbm, out_hbm, *refs):
        sidx = refs[0:NR]
        didx = refs[NR:2 * NR]
        rows = refs[2 * NR:3 * NR]
        agg_sh = refs[3 * NR]
        isem = refs[3 * NR + 1:3 * NR + 1 + NR]
        gsem = refs[3 * NR + 1 + NR:3 * NR + 1 + 2 * NR]
        ssem = refs[3 * NR + 1 + 2 * NR:3 * NR + 1 + 3 * NR]
        c = lax.axis_index("c")
        s = lax.axis_index("s")
        wid = c * NS + s
        base_n = s * RPT
        pltpu.sync_copy(zeros_hbm.at[pl.ds(base_n, RPT)],
                        agg_sh.at[pl.ds(base_n, RPT)])

        def start_idx(j, m):
            base_e = wid * EPW + jnp.minimum(j, nch - 1) * k
            pltpu.async_copy(src_hbm.at[pl.ds(base_e, k)], sidx[m], isem[m])
            pltpu.async_copy(dst_hbm.at[pl.ds(base_e, k)], didx[m], isem[m])

        def wait_idx(m):
            pltpu.make_async_copy(src_hbm.at[pl.ds(0, k)], sidx[m],
                                  isem[m]).wait()
            pltpu.make_async_copy(dst_hbm.at[pl.ds(0, k)], didx[m],
                                  isem[m]).wait()

        def start_gather(m):
            pltpu.async_copy(h_hbm.at[sidx[m]], rows[m], gsem[m])

        def wait_gather(m):
            pltpu.make_async_copy(h_hbm.at[pl.ds(0, k)], rows[m],
                                  gsem[m]).wait()

        def start_scatter(m):
            pltpu.async_copy(rows[m], agg_sh.at[didx[m]], ssem[m], add=True)

        def wait_scatter(m):
            pltpu.make_async_copy(h_hbm.at[pl.ds(0, k)], rows[m],
                                  ssem[m]).wait()

        def phase(j, p):
            p1 = (p + 1) % NR
            p2 = (p + 2) % NR
            wait_gather(p)
            if not (isinstance(j, int) and j < 2):
                wait_scatter(p2)
            wait_idx(p1)
            start_gather(p1)
            start_idx(j + 2, p2)
            start_scatter(p)

        plsc.subcore_barrier()
        start_idx(0, 0)
        start_idx(1, 1)
        wait_idx(0)
        start_gather(0)
        for j in range(NR):
            phase(j, j)

        def body(g, carry):
            for p in range(NR):
                phase(NR + g * NR + p, p)
            return carry

        lax.fori_loop(0, (nch - NR) // NR, body, 0)
        wait_gather(nch % NR)
        wait_idx((nch + 1) % NR)
        wait_scatter((nch - 2) % NR)
        wait_scatter((nch - 1) % NR)
        plsc.subcore_barrier()
        pltpu.sync_copy(agg_sh.at[pl.ds(base_n, RPT)],
                        out_hbm.at[c, pl.ds(base_n, RPT)])

    return agg_kernel


BLK = 2048


def _xw_body(x_ref, w_ref, xw_ref):
    xw_ref[...] = jnp.dot(x_ref[...], w_ref[...],
                          preferred_element_type=jnp.float32)


def _tc_xw(x_pad, W1):
    return pl.pallas_call(
        _xw_body,
        grid=(NP // BLK,),
        in_specs=[
            pl.BlockSpec((BLK, D_IN), lambda i: (i, 0)),
            pl.BlockSpec((D_IN, D_H), lambda i: (0, 0)),
        ],
        out_specs=pl.BlockSpec((BLK, D_H), lambda i: (i, 0)),
        out_shape=jax.ShapeDtypeStruct((NP, D_H), jnp.float32),
    )(x_pad, W1)


def _layer1_body(dout_ref, din_ref, xw_ref, so_ref, si_ref, h_ref):
    deg_out = jnp.maximum(dout_ref[0, :] + dout_ref[1, :], 1.0)
    deg_in = jnp.maximum(din_ref[0, :] + din_ref[1, :], 1.0)
    so = lax.rsqrt(deg_out)
    si = lax.rsqrt(deg_in)
    so_ref[...] = so
    si_ref[...] = si
    h_ref[...] = (xw_ref[...] * so[:, None]).astype(jnp.bfloat16)


def _tc_layer1(dout_p, din_p, xw, W1):
    return pl.pallas_call(
        _layer1_body,
        grid=(NP // BLK,),
        in_specs=[
            pl.BlockSpec((NC, BLK), lambda i: (0, i)),
            pl.BlockSpec((NC, BLK), lambda i: (0, i)),
            pl.BlockSpec((BLK, D_H), lambda i: (i, 0)),
        ],
        out_specs=[
            pl.BlockSpec((BLK,), lambda i: (i,)),
            pl.BlockSpec((BLK,), lambda i: (i,)),
            pl.BlockSpec((BLK, D_H), lambda i: (i, 0)),
        ],
        out_shape=[
            jax.ShapeDtypeStruct((NP,), jnp.float32),
            jax.ShapeDtypeStruct((NP,), jnp.float32),
            jax.ShapeDtypeStruct((NP, D_H), jnp.bfloat16),
        ],
    )(dout_p, din_p, xw)


def _layer2_body(agg_ref, si_ref, so_ref, b_ref, w_ref, h_ref):
    agg = (agg_ref[0, :, :].astype(jnp.float32)
           + agg_ref[1, :, :].astype(jnp.float32))
    h = agg * si_ref[...][:, None] + b_ref[...][None, :]
    h = jnp.maximum(h, 0.0)
    h_ref[...] = jnp.dot(h * so_ref[...][:, None], w_ref[...],
                         preferred_element_type=jnp.float32
                         ).astype(jnp.bfloat16)


def _tc_layer2(aggp1, si, so, b1, W2):
    return pl.pallas_call(
        _layer2_body,
        grid=(NP // BLK,),
        in_specs=[
            pl.BlockSpec((NC, BLK, D_H), lambda i: (0, i, 0)),
            pl.BlockSpec((BLK,), lambda i: (i,)),
            pl.BlockSpec((BLK,), lambda i: (i,)),
            pl.BlockSpec((D_H,), lambda i: (0,)),
            pl.BlockSpec((D_H, D_OUT), lambda i: (0, 0)),
        ],
        out_specs=pl.BlockSpec((BLK, D_OUT), lambda i: (i, 0)),
        out_shape=jax.ShapeDtypeStruct((NP, D_OUT), jnp.bfloat16),
    )(aggp1, si, so, b1, W2)


def _final_body(agg_ref, si_ref, b_ref, out_ref):
    agg = (agg_ref[0, :, :].astype(jnp.float32)
           + agg_ref[1, :, :].astype(jnp.float32))
    out_ref[...] = agg * si_ref[...][:, None] + b_ref[...][None, :]


def _tc_final(aggp2, si, b2):
    return pl.pallas_call(
        _final_body,
        grid=(NP // BLK,),
        in_specs=[
            pl.BlockSpec((NC, BLK, D_OUT), lambda i: (0, i, 0)),
            pl.BlockSpec((BLK,), lambda i: (i,)),
            pl.BlockSpec((D_OUT,), lambda i: (0,)),
        ],
        out_specs=pl.BlockSpec((BLK, D_OUT), lambda i: (i, 0)),
        out_shape=jax.ShapeDtypeStruct((NP, D_OUT), jnp.float32),
    )(aggp2, si, b2)


def kernel(features, edge_index, W1, b1, W2, b2):
    pad_nodes = N + jax.lax.rem(jnp.arange(EP - E, dtype=jnp.int32),
                                jnp.int32(NP - N))
    epad = jnp.stack([pad_nodes, pad_nodes])
    ei = jnp.concatenate([edge_index, epad], axis=1)
    src = ei[0]
    dst = ei[1]
    x_pad = jnp.pad(features, ((0, NP - N), (0, 0)))
    zeros_n = jnp.zeros((NP,), jnp.float32)
    zeros_h = jnp.zeros((NP, D_H), jnp.bfloat16)
    zeros_o = jnp.zeros((NP, D_OUT), jnp.bfloat16)

    xw = _tc_xw(x_pad, W1)
    dout_p, din_p = _make_deg_kernel()(src, dst, zeros_n)
    so, si, h1 = _tc_layer1(dout_p, din_p, xw, W1)
    aggp1 = _make_agg_kernel(D_H, 256)(src, dst, h1, zeros_h)
    h2 = _tc_layer2(aggp1, si, so, b1, W2)
    aggp2 = _make_agg_kernel(D_OUT, 512)(src, dst, h2, zeros_o)
    return _tc_final(aggp2, si, b2)[:N]


# speedup vs baseline: 3.4220x; 1.0115x over previous
import functools

import jax
import jax.numpy as jnp
from jax import lax
from jax.experimental import pallas as pl
from jax.experimental.pallas import tpu as pltpu
from jax.experimental.pallas import tpu_sc as plsc

N = 10000
NP = 10240
E = 320000
EP = 327680
D_IN = 128
D_H = 128
D_OUT = 64

NC = 2
NS = 16
NW = NC * NS
EPW = EP // NW
RPT = NP // NS
NR = 4

NP2 = 10112
RPT2 = NP2 // NS
AGG_K = 128
AGG_NCH = EPW // AGG_K
NB = 3


def _sc_mesh():
    return plsc.VectorSubcoreMesh(core_axis_name="c", subcore_axis_name="s")


_SC_PARAMS = pltpu.CompilerParams(use_tc_tiling_on_sc=False)


DEG_K = 256
DEG_NCH = EPW // DEG_K


def _make_deg_kernel():
    @functools.partial(
        pl.kernel,
        out_type=(
            jax.ShapeDtypeStruct((NC, NP), jnp.float32),
            jax.ShapeDtypeStruct((NC, NP), jnp.float32),
        ),
        mesh=_sc_mesh(),
        scratch_types=(
            [pltpu.VMEM((DEG_K,), jnp.int32) for _ in range(2 * NR)]
            + [pltpu.VMEM((DEG_K,), jnp.float32),
               pltpu.VMEM_SHARED((NP,), jnp.float32),
               pltpu.VMEM_SHARED((NP,), jnp.float32)]
            + [pltpu.SemaphoreType.DMA for _ in range(NR + 1)]
        ),
        compiler_params=_SC_PARAMS,
    )
    def deg_kernel(src_hbm, dst_hbm, zeros_hbm, dout_hbm, din_hbm, *refs):
        sidx = refs[0:NR]
        didx = refs[NR:2 * NR]
        ones_v, dsrc_sh, ddst_sh = refs[2 * NR:2 * NR + 3]
        isem = refs[2 * NR + 3:2 * NR + 3 + NR]
        ssem = refs[2 * NR + 3 + NR]
        c = lax.axis_index("c")
        s = lax.axis_index("s")
        wid = c * NS + s
        base_n = s * RPT
        pltpu.sync_copy(zeros_hbm.at[pl.ds(base_n, RPT)],
                        dsrc_sh.at[pl.ds(base_n, RPT)])
        pltpu.sync_copy(zeros_hbm.at[pl.ds(base_n, RPT)],
                        ddst_sh.at[pl.ds(base_n, RPT)])
        for i in range(DEG_K // 16):
            ones_v[pl.ds(i * 16, 16)] = jnp.ones((16,), jnp.float32)
        plsc.subcore_barrier()

        def start_idx(j, m):
            base_e = wid * EPW + jnp.minimum(j, DEG_NCH - 1) * DEG_K
            pltpu.async_copy(src_hbm.at[pl.ds(base_e, DEG_K)], sidx[m],
                             isem[m])
            pltpu.async_copy(dst_hbm.at[pl.ds(base_e, DEG_K)], didx[m],
                             isem[m])

        def wait_idx(m):
            pltpu.make_async_copy(src_hbm.at[pl.ds(0, DEG_K)], sidx[m],
                                  isem[m]).wait()
            pltpu.make_async_copy(dst_hbm.at[pl.ds(0, DEG_K)], didx[m],
                                  isem[m]).wait()

        start_idx(0, 0)
        start_idx(1, 1)

        def body(g, carry):
            for p in range(NR):
                j = g * NR + p
                wait_idx(p)
                start_idx(j + 2, (p + 2) % NR)
                pltpu.async_copy(ones_v, dsrc_sh.at[sidx[p]], ssem, add=True)
                pltpu.sync_copy(ones_v, ddst_sh.at[didx[p]], add=True)
                pltpu.make_async_copy(zeros_hbm.at[pl.ds(0, DEG_K)], ones_v,
                                      ssem).wait()
            return carry

        lax.fori_loop(0, DEG_NCH // NR, body, 0)
        wait_idx(DEG_NCH % NR)
        wait_idx((DEG_NCH + 1) % NR)
        plsc.subcore_barrier()
        pltpu.sync_copy(dsrc_sh.at[pl.ds(base_n, RPT)],
                        dout_hbm.at[c, pl.ds(base_n, RPT)])
        pltpu.sync_copy(ddst_sh.at[pl.ds(base_n, RPT)],
                        din_hbm.at[c, pl.ds(base_n, RPT)])

    return deg_kernel


def _make_agg_kernel(d, k):
    nch = EPW // k
    assert nch % NR == 0

    @functools.partial(
        pl.kernel,
        out_type=jax.ShapeDtypeStruct((NC, NP, d), jnp.bfloat16),
        mesh=_sc_mesh(),
        scratch_types=(
            [pltpu.VMEM((k,), jnp.int32) for _ in range(2 * NR)]
            + [pltpu.VMEM((k, d), jnp.bfloat16) for _ in range(NR)]
            + [pltpu.VMEM_SHARED((NP, d), jnp.bfloat16)]
            + [pltpu.SemaphoreType.DMA for _ in range(3 * NR)]
        ),
        compiler_params=_SC_PARAMS,
    )
    def agg_kernel(src_hbm, dst_hbm, h_hbm, zeros_hbm, out_hbm, *refs):
        sidx = refs[0:NR]
        didx = refs[NR:2 * NR]
        rows = refs[2 * NR:3 * NR]
        agg_sh = refs[3 * NR]
        isem = refs[3 * NR + 1:3 * NR + 1 + NR]
        gsem = refs[3 * NR + 1 + NR:3 * NR + 1 + 2 * NR]
        ssem = refs[3 * NR + 1 + 2 * NR:3 * NR + 1 + 3 * NR]
        c = lax.axis_index("c")
        s = lax.axis_index("s")
        wid = c * NS + s
        base_n = s * RPT
        pltpu.sync_copy(zeros_hbm.at[pl.ds(base_n, RPT)],
                        agg_sh.at[pl.ds(base_n, RPT)])

        def start_idx(j, m):
            base_e = wid * EPW + jnp.minimum(j, nch - 1) * k
            pltpu.async_copy(src_hbm.at[pl.ds(base_e, k)], sidx[m], isem[m])
            pltpu.async_copy(dst_hbm.at[pl.ds(base_e, k)], didx[m], isem[m])

        def wait_idx(m):
            pltpu.make_async_copy(src_hbm.at[pl.ds(0, k)], sidx[m],
                                  isem[m]).wait()
            pltpu.make_async_copy(dst_hbm.at[pl.ds(0, k)], didx[m],
                                  isem[m]).wait()

        def start_gather(m):
            pltpu.async_copy(h_hbm.at[sidx[m]], rows[m], gsem[m])

        def wait_gather(m):
            pltpu.make_async_copy(h_hbm.at[pl.ds(0, k)], rows[m],
                                  gsem[m]).wait()

        def start_scatter(m):
            pltpu.async_copy(rows[m], agg_sh.at[didx[m]], ssem[m], add=True)

        def wait_scatter(m):
            pltpu.make_async_copy(h_hbm.at[pl.ds(0, k)], rows[m],
                                  ssem[m]).wait()

        def phase(j, p):
            p1 = (p + 1) % NR
            p2 = (p + 2) % NR
            wait_gather(p)
            if not (isinstance(j, int) and j < 2):
                wait_scatter(p2)
            wait_idx(p1)
            start_gather(p1)
            start_idx(j + 2, p2)
            start_scatter(p)

        plsc.subcore_barrier()
        start_idx(0, 0)
        start_idx(1, 1)
        wait_idx(0)
        start_gather(0)
        for j in range(NR):
            phase(j, j)

        def body(g, carry):
            for p in range(NR):
                phase(NR + g * NR + p, p)
            return carry

        lax.fori_loop(0, (nch - NR) // NR, body, 0)
        wait_gather(nch % NR)
        wait_idx((nch + 1) % NR)
        wait_scatter((nch - 2) % NR)
        wait_scatter((nch - 1) % NR)
        plsc.subcore_barrier()
        pltpu.sync_copy(agg_sh.at[pl.ds(base_n, RPT)],
                        out_hbm.at[c, pl.ds(base_n, RPT)])

    return agg_kernel


BLK = 2048


def _xw_body(x_ref, w_ref, xw_ref):
    xw_ref[...] = jnp.dot(x_ref[...], w_ref[...],
                          preferred_element_type=jnp.float32)


def _tc_xw(x_pad, W1):
    return pl.pallas_call(
        _xw_body,
        grid=(NP // BLK,),
        in_specs=[
            pl.BlockSpec((BLK, D_IN), lambda i: (i, 0)),
            pl.BlockSpec((D_IN, D_H), lambda i: (0, 0)),
        ],
        out_specs=pl.BlockSpec((BLK, D_H), lambda i: (i, 0)),
        out_shape=jax.ShapeDtypeStruct((NP, D_H), jnp.float32),
    )(x_pad, W1)


def _layer1_body(dout_ref, din_ref, xw_ref, so_ref, si_ref, h_ref):
    deg_out = jnp.maximum(dout_ref[0, :] + dout_ref[1, :], 1.0)
    deg_in = jnp.maximum(din_ref[0, :] + din_ref[1, :], 1.0)
    so = lax.rsqrt(deg_out)
    si = lax.rsqrt(deg_in)
    so_ref[...] = so
    si_ref[...] = si
    h_ref[...] = (xw_ref[...] * so[:, None]).astype(jnp.bfloat16)


def _tc_layer1(dout_p, din_p, xw, W1):
    return pl.pallas_call(
        _layer1_body,
        grid=(NP // BLK,),
        in_specs=[
            pl.BlockSpec((NC, BLK), lambda i: (0, i)),
            pl.BlockSpec((NC, BLK), lambda i: (0, i)),
            pl.BlockSpec((BLK, D_H), lambda i: (i, 0)),
        ],
        out_specs=[
            pl.BlockSpec((BLK,), lambda i: (i,)),
            pl.BlockSpec((BLK,), lambda i: (i,)),
            pl.BlockSpec((BLK, D_H), lambda i: (i, 0)),
        ],
        out_shape=[
            jax.ShapeDtypeStruct((NP,), jnp.float32),
            jax.ShapeDtypeStruct((NP,), jnp.float32),
            jax.ShapeDtypeStruct((NP, D_H), jnp.bfloat16),
        ],
    )(dout_p, din_p, xw)


def _layer2_body(agg_ref, si_ref, so_ref, b_ref, w_ref, h_ref):
    agg = (agg_ref[0, :, :].astype(jnp.float32)
           + agg_ref[1, :, :].astype(jnp.float32))
    h = agg * si_ref[...][:, None] + b_ref[...][None, :]
    h = jnp.maximum(h, 0.0)
    h_ref[...] = jnp.dot(h * so_ref[...][:, None], w_ref[...],
                         preferred_element_type=jnp.float32
                         ).astype(jnp.bfloat16)


def _tc_layer2(aggp1, si, so, b1, W2):
    return pl.pallas_call(
        _layer2_body,
        grid=(NP // BLK,),
        in_specs=[
            pl.BlockSpec((NC, BLK, D_H), lambda i: (0, i, 0)),
            pl.BlockSpec((BLK,), lambda i: (i,)),
            pl.BlockSpec((BLK,), lambda i: (i,)),
            pl.BlockSpec((D_H,), lambda i: (0,)),
            pl.BlockSpec((D_H, D_OUT), lambda i: (0, 0)),
        ],
        out_specs=pl.BlockSpec((BLK, D_OUT), lambda i: (i, 0)),
        out_shape=jax.ShapeDtypeStruct((NP, D_OUT), jnp.bfloat16),
    )(aggp1, si, so, b1, W2)


def _final_body(agg_ref, si_ref, b_ref, out_ref):
    agg = (agg_ref[0, :, :].astype(jnp.float32)
           + agg_ref[1, :, :].astype(jnp.float32))
    out_ref[...] = agg * si_ref[...][:, None] + b_ref[...][None, :]


def _tc_final(aggp2, si, b2):
    return pl.pallas_call(
        _final_body,
        grid=(NP // BLK,),
        in_specs=[
            pl.BlockSpec((NC, BLK, D_OUT), lambda i: (0, i, 0)),
            pl.BlockSpec((BLK,), lambda i: (i,)),
            pl.BlockSpec((D_OUT,), lambda i: (0,)),
        ],
        out_specs=pl.BlockSpec((BLK, D_OUT), lambda i: (i, 0)),
        out_shape=jax.ShapeDtypeStruct((NP, D_OUT), jnp.float32),
    )(aggp2, si, b2)


def kernel(features, edge_index, W1, b1, W2, b2):
    pad_nodes = N + jax.lax.rem(jnp.arange(EP - E, dtype=jnp.int32),
                                jnp.int32(NP - N))
    epad = jnp.stack([pad_nodes, pad_nodes])
    ei = jnp.concatenate([edge_index, epad], axis=1)
    src = ei[0]
    dst = ei[1]
    x_pad = jnp.pad(features, ((0, NP - N), (0, 0)))
    zeros_n = jnp.zeros((NP,), jnp.float32)
    zeros_h = jnp.zeros((NP, D_H), jnp.bfloat16)
    zeros_o = jnp.zeros((NP, D_OUT), jnp.bfloat16)

    xw = _tc_xw(x_pad, W1)
    dout_p, din_p = _make_deg_kernel()(src, dst, zeros_n)
    so, si, h1 = _tc_layer1(dout_p, din_p, xw, W1)
    aggp1 = _make_agg_kernel(D_H, 320)(src, dst, h1, zeros_h)
    h2 = _tc_layer2(aggp1, si, so, b1, W2)
    aggp2 = _make_agg_kernel(D_OUT, 512)(src, dst, h2, zeros_o)
    return _tc_final(aggp2, si, b2)[:N]


# speedup vs baseline: 3.4239x; 1.0006x over previous
import functools

import jax
import jax.numpy as jnp
from jax import lax
from jax.experimental import pallas as pl
from jax.experimental.pallas import tpu as pltpu
from jax.experimental.pallas import tpu_sc as plsc

N = 10000
NP = 10240
E = 320000
EP = 327680
D_IN = 128
D_H = 128
D_OUT = 64

NC = 2
NS = 16
NW = NC * NS
EPW = EP // NW
RPT = NP // NS
NR = 4


def _sc_mesh():
    return plsc.VectorSubcoreMesh(core_axis_name="c", subcore_axis_name="s")


_SC_PARAMS = pltpu.CompilerParams(use_tc_tiling_on_sc=False)


DEG_K = 256
DEG_NCH = EPW // DEG_K


def _make_deg_kernel():
    @functools.partial(
        pl.kernel,
        out_type=(
            jax.ShapeDtypeStruct((NC, NP), jnp.float32),
            jax.ShapeDtypeStruct((NC, NP), jnp.float32),
        ),
        mesh=_sc_mesh(),
        scratch_types=(
            [pltpu.VMEM((DEG_K,), jnp.int32) for _ in range(2 * NR)]
            + [pltpu.VMEM((DEG_K,), jnp.float32),
               pltpu.VMEM_SHARED((NP,), jnp.float32),
               pltpu.VMEM_SHARED((NP,), jnp.float32)]
            + [pltpu.SemaphoreType.DMA for _ in range(NR + 1)]
        ),
        compiler_params=_SC_PARAMS,
    )
    def deg_kernel(src_hbm, dst_hbm, zeros_hbm, dout_hbm, din_hbm, *refs):
        sidx = refs[0:NR]
        didx = refs[NR:2 * NR]
        ones_v, dsrc_sh, ddst_sh = refs[2 * NR:2 * NR + 3]
        isem = refs[2 * NR + 3:2 * NR + 3 + NR]
        ssem = refs[2 * NR + 3 + NR]
        c = lax.axis_index("c")
        s = lax.axis_index("s")
        wid = c * NS + s
        base_n = s * RPT
        pltpu.sync_copy(zeros_hbm.at[pl.ds(base_n, RPT)],
                        dsrc_sh.at[pl.ds(base_n, RPT)])
        pltpu.sync_copy(zeros_hbm.at[pl.ds(base_n, RPT)],
                        ddst_sh.at[pl.ds(base_n, RPT)])
        for i in range(DEG_K // 16):
            ones_v[pl.ds(i * 16, 16)] = jnp.ones((16,), jnp.float32)
        plsc.subcore_barrier()

        def start_idx(j, m):
            base_e = wid * EPW + jnp.minimum(j, DEG_NCH - 1) * DEG_K
            pltpu.async_copy(src_hbm.at[pl.ds(base_e, DEG_K)], sidx[m],
                             isem[m])
            pltpu.async_copy(dst_hbm.at[pl.ds(base_e, DEG_K)], didx[m],
                             isem[m])

        def wait_idx(m):
            pltpu.make_async_copy(src_hbm.at[pl.ds(0, DEG_K)], sidx[m],
                                  isem[m]).wait()
            pltpu.make_async_copy(dst_hbm.at[pl.ds(0, DEG_K)], didx[m],
                                  isem[m]).wait()

        start_idx(0, 0)
        start_idx(1, 1)

        def body(g, carry):
            for p in range(NR):
                j = g * NR + p
                wait_idx(p)
                start_idx(j + 2, (p + 2) % NR)
                pltpu.async_copy(ones_v, dsrc_sh.at[sidx[p]], ssem, add=True)
                pltpu.sync_copy(ones_v, ddst_sh.at[didx[p]], add=True)
                pltpu.make_async_copy(zeros_hbm.at[pl.ds(0, DEG_K)], ones_v,
                                      ssem).wait()
            return carry

        lax.fori_loop(0, DEG_NCH // NR, body, 0)
        wait_idx(DEG_NCH % NR)
        wait_idx((DEG_NCH + 1) % NR)
        plsc.subcore_barrier()
        pltpu.sync_copy(dsrc_sh.at[pl.ds(base_n, RPT)],
                        dout_hbm.at[c, pl.ds(base_n, RPT)])
        pltpu.sync_copy(ddst_sh.at[pl.ds(base_n, RPT)],
                        din_hbm.at[c, pl.ds(base_n, RPT)])

    return deg_kernel


def _make_agg_kernel(d, k):
    nch = EPW // k
    assert nch % NR == 0

    @functools.partial(
        pl.kernel,
        out_type=jax.ShapeDtypeStruct((NC, NP, d), jnp.bfloat16),
        mesh=_sc_mesh(),
        scratch_types=(
            [pltpu.VMEM((k,), jnp.int32) for _ in range(2 * NR)]
            + [pltpu.VMEM((k, d), jnp.bfloat16) for _ in range(NR)]
            + [pltpu.VMEM_SHARED((NP, d), jnp.bfloat16)]
            + [pltpu.SemaphoreType.DMA for _ in range(3 * NR)]
        ),
        compiler_params=_SC_PARAMS,
    )
    def agg_kernel(src_hbm, dst_hbm, h_hbm, zeros_hbm, out_hbm, *refs):
        sidx = refs[0:NR]
        didx = refs[NR:2 * NR]
        rows = refs[2 * NR:3 * NR]
        agg_sh = refs[3 * NR]
        isem = refs[3 * NR + 1:3 * NR + 1 + NR]
        gsem = refs[3 * NR + 1 + NR:3 * NR + 1 + 2 * NR]
        ssem = refs[3 * NR + 1 + 2 * NR:3 * NR + 1 + 3 * NR]
        c = lax.axis_index("c")
        s = lax.axis_index("s")
        wid = c * NS + s
        base_n = s * RPT
        pltpu.sync_copy(zeros_hbm.at[pl.ds(base_n, RPT)],
                        agg_sh.at[pl.ds(base_n, RPT)])

        def start_idx(j, m):
            base_e = wid * EPW + jnp.minimum(j, nch - 1) * k
            pltpu.async_copy(src_hbm.at[pl.ds(base_e, k)], sidx[m], isem[m])
            pltpu.async_copy(dst_hbm.at[pl.ds(base_e, k)], didx[m], isem[m])

        def wait_idx(m):
            pltpu.make_async_copy(src_hbm.at[pl.ds(0, k)], sidx[m],
                                  isem[m]).wait()
            pltpu.make_async_copy(dst_hbm.at[pl.ds(0, k)], didx[m],
                                  isem[m]).wait()

        def start_gather(m):
            pltpu.async_copy(h_hbm.at[sidx[m]], rows[m], gsem[m])

        def wait_gather(m):
            pltpu.make_async_copy(h_hbm.at[pl.ds(0, k)], rows[m],
                                  gsem[m]).wait()

        def start_scatter(m):
            pltpu.async_copy(rows[m], agg_sh.at[didx[m]], ssem[m], add=True)

        def wait_scatter(m):
            pltpu.make_async_copy(h_hbm.at[pl.ds(0, k)], rows[m],
                                  ssem[m]).wait()

        def phase(j, p):
            p1 = (p + 1) % NR
            p2 = (p + 2) % NR
            wait_gather(p)
            if not (isinstance(j, int) and j < 2):
                wait_scatter(p2)
            wait_idx(p1)
            start_gather(p1)
            start_idx(j + 2, p2)
            start_scatter(p)

        plsc.subcore_barrier()
        start_idx(0, 0)
        start_idx(1, 1)
        wait_idx(0)
        start_gather(0)
        for j in range(NR):
            phase(j, j)

        def body(g, carry):
            for p in range(NR):
                phase(NR + g * NR + p, p)
            return carry

        lax.fori_loop(0, (nch - NR) // NR, body, 0)
        wait_gather(nch % NR)
        wait_idx((nch + 1) % NR)
        wait_scatter((nch - 2) % NR)
        wait_scatter((nch - 1) % NR)
        plsc.subcore_barrier()
        pltpu.sync_copy(agg_sh.at[pl.ds(base_n, RPT)],
                        out_hbm.at[c, pl.ds(base_n, RPT)])

    return agg_kernel


BLK = 2048


def _xw_body(x_ref, w_ref, xw_ref):
    xw_ref[...] = jnp.dot(x_ref[...], w_ref[...],
                          preferred_element_type=jnp.float32)


def _tc_xw(x_pad, W1):
    return pl.pallas_call(
        _xw_body,
        grid=(NP // BLK,),
        in_specs=[
            pl.BlockSpec((BLK, D_IN), lambda i: (i, 0)),
            pl.BlockSpec((D_IN, D_H), lambda i: (0, 0)),
        ],
        out_specs=pl.BlockSpec((BLK, D_H), lambda i: (i, 0)),
        out_shape=jax.ShapeDtypeStruct((NP, D_H), jnp.float32),
    )(x_pad, W1)


def _layer1_body(dout_ref, din_ref, xw_ref, so_ref, si_ref, h_ref):
    deg_out = jnp.maximum(dout_ref[0, :] + dout_ref[1, :], 1.0)
    deg_in = jnp.maximum(din_ref[0, :] + din_ref[1, :], 1.0)
    so = lax.rsqrt(deg_out)
    si = lax.rsqrt(deg_in)
    so_ref[...] = so
    si_ref[...] = si
    h_ref[...] = (xw_ref[...] * so[:, None]).astype(jnp.bfloat16)


def _tc_layer1(dout_p, din_p, xw):
    return pl.pallas_call(
        _layer1_body,
        grid=(NP // BLK,),
        in_specs=[
            pl.BlockSpec((NC, BLK), lambda i: (0, i)),
            pl.BlockSpec((NC, BLK), lambda i: (0, i)),
            pl.BlockSpec((BLK, D_H), lambda i: (i, 0)),
        ],
        out_specs=[
            pl.BlockSpec((BLK,), lambda i: (i,)),
            pl.BlockSpec((BLK,), lambda i: (i,)),
            pl.BlockSpec((BLK, D_H), lambda i: (i, 0)),
        ],
        out_shape=[
            jax.ShapeDtypeStruct((NP,), jnp.float32),
            jax.ShapeDtypeStruct((NP,), jnp.float32),
            jax.ShapeDtypeStruct((NP, D_H), jnp.bfloat16),
        ],
    )(dout_p, din_p, xw)


def _layer2_body(agg_ref, si_ref, so_ref, b_ref, w_ref, h_ref):
    agg = (agg_ref[0, :, :].astype(jnp.float32)
           + agg_ref[1, :, :].astype(jnp.float32))
    h = agg * si_ref[...][:, None] + b_ref[...][None, :]
    h = jnp.maximum(h, 0.0)
    h_ref[...] = jnp.dot(h * so_ref[...][:, None], w_ref[...],
                         preferred_element_type=jnp.float32
                         ).astype(jnp.bfloat16)


def _tc_layer2(aggp1, si, so, b1, W2):
    return pl.pallas_call(
        _layer2_body,
        grid=(NP // BLK,),
        in_specs=[
            pl.BlockSpec((NC, BLK, D_H), lambda i: (0, i, 0)),
            pl.BlockSpec((BLK,), lambda i: (i,)),
            pl.BlockSpec((BLK,), lambda i: (i,)),
            pl.BlockSpec((D_H,), lambda i: (0,)),
            pl.BlockSpec((D_H, D_OUT), lambda i: (0, 0)),
        ],
        out_specs=pl.BlockSpec((BLK, D_OUT), lambda i: (i, 0)),
        out_shape=jax.ShapeDtypeStruct((NP, D_OUT), jnp.bfloat16),
    )(aggp1, si, so, b1, W2)


def _final_body(agg_ref, si_ref, b_ref, out_ref):
    agg = (agg_ref[0, :, :].astype(jnp.float32)
           + agg_ref[1, :, :].astype(jnp.float32))
    out_ref[...] = agg * si_ref[...][:, None] + b_ref[...][None, :]


def _tc_final(aggp2, si, b2):
    return pl.pallas_call(
        _final_body,
        grid=(NP // BLK,),
        in_specs=[
            pl.BlockSpec((NC, BLK, D_OUT), lambda i: (0, i, 0)),
            pl.BlockSpec((BLK,), lambda i: (i,)),
            pl.BlockSpec((D_OUT,), lambda i: (0,)),
        ],
        out_specs=pl.BlockSpec((BLK, D_OUT), lambda i: (i, 0)),
        out_shape=jax.ShapeDtypeStruct((NP, D_OUT), jnp.float32),
    )(aggp2, si, b2)


def kernel(features, edge_index, W1, b1, W2, b2):
    pad_nodes = N + jax.lax.rem(jnp.arange(EP - E, dtype=jnp.int32),
                                jnp.int32(NP - N))
    epad = jnp.stack([pad_nodes, pad_nodes])
    ei = jnp.concatenate([edge_index, epad], axis=1)
    src = ei[0]
    dst = ei[1]
    x_pad = jnp.pad(features, ((0, NP - N), (0, 0)))
    zeros_n = jnp.zeros((NP,), jnp.float32)
    zeros_h = jnp.zeros((NP, D_H), jnp.bfloat16)
    zeros_o = jnp.zeros((NP, D_OUT), jnp.bfloat16)

    xw = _tc_xw(x_pad, W1)
    dout_p, din_p = _make_deg_kernel()(src, dst, zeros_n)
    so, si, h1 = _tc_layer1(dout_p, din_p, xw)
    aggp1 = _make_agg_kernel(D_H, 320)(src, dst, h1, zeros_h)
    h2 = _tc_layer2(aggp1, si, so, b1, W2)
    aggp2 = _make_agg_kernel(D_OUT, 512)(src, dst, h2, zeros_o)
    return _tc_final(aggp2, si, b2)[:N]
